# Initial kernel scaffold; baseline (speedup 1.0000x reference)
import jax
import jax.numpy as jnp
from jax.experimental import pallas as pl


def kernel(x, edge_index, W1, b1, W2, b2):
    raise NotImplementedError("write your pallas kernel here")


# speedup vs baseline: 11.9035x; 11.9035x over previous
import functools

import jax
import jax.numpy as jnp
from jax import lax
from jax.experimental import pallas as pl
from jax.experimental.pallas import tpu as pltpu
from jax.experimental.pallas import tpu_sc as plsc

N = 10000
N_PAD = 10240
E = 160000
D_IN = 256
D_HID = 256
D_OUT = 64

NC = 2
NS = 16
TPN = N_PAD // NS
ZR = 64

MP_K = 125
MP_NB = (E // NS) // MP_K

DEG_W = 128

_mesh = plsc.VectorSubcoreMesh(core_axis_name="c", subcore_axis_name="s")


def _sc_degree(dst2d):
    wnb = MP_NB // NC

    @functools.partial(
        pl.kernel,
        out_type=jax.ShapeDtypeStruct((NC * N_PAD, DEG_W), jnp.float32),
        mesh=_mesh,
        scratch_types=[
            pltpu.VMEM((wnb, MP_K), jnp.int32),
            pltpu.VMEM((MP_K, DEG_W), jnp.float32),
            pltpu.VMEM((ZR, DEG_W), jnp.float32),
            pltpu.VMEM_SHARED((N_PAD, DEG_W), jnp.float32),
        ],
    )
    def k(dst_hbm, out_hbm, dstv, ones, zbuf, hist_sh):
        c = lax.axis_index("c")
        s = lax.axis_index("s")
        w = c * NS + s

        def fill_ones(r, _):
            def col(j, _):
                ones[r, pl.ds(j * 16, 16)] = jnp.ones((16,), jnp.float32)
                return 0
            return lax.fori_loop(0, DEG_W // 16, col, 0)

        lax.fori_loop(0, MP_K, fill_ones, 0)

        def fill_zero(r, _):
            def col(j, _):
                zbuf[r, pl.ds(j * 16, 16)] = jnp.zeros((16,), jnp.float32)
                return 0
            return lax.fori_loop(0, DEG_W // 16, col, 0)

        lax.fori_loop(0, ZR, fill_zero, 0)

        def zero_blk(j, _):
            pltpu.sync_copy(zbuf, hist_sh.at[pl.ds(s * TPN + j * ZR, ZR)])
            return 0

        lax.fori_loop(0, TPN // ZR, zero_blk, 0)
        pltpu.sync_copy(dst_hbm.at[pl.ds(w * wnb, wnb)], dstv)
        plsc.subcore_barrier()

        def scat(b, _):
            pltpu.sync_copy(ones, hist_sh.at[dstv.at[b]], add=True)
            return 0

        lax.fori_loop(0, wnb, scat, 0)
        plsc.subcore_barrier()

        def wout(j, _):
            pltpu.sync_copy(hist_sh.at[pl.ds(s * TPN + j * ZR, ZR)], zbuf)
            base = pl.multiple_of(c * N_PAD + s * TPN + j * ZR, ZR)
            pltpu.sync_copy(zbuf, out_hbm.at[pl.ds(base, ZR)])
            return 0

        lax.fori_loop(0, TPN // ZR, wout, 0)

    return k(dst2d)


def _sc_msgpass(hp_lo, hp_hi, src2d, dst2d, dh):
    @functools.partial(
        pl.kernel,
        out_type=[jax.ShapeDtypeStruct((N_PAD, dh), jnp.float32)] * 2,
        mesh=_mesh,
        scratch_types=[
            pltpu.VMEM((MP_NB, MP_K), jnp.int32),
            pltpu.VMEM((MP_NB, MP_K), jnp.int32),
            pltpu.VMEM((MP_K, dh), jnp.float32),
            pltpu.VMEM((ZR, dh), jnp.float32),
            pltpu.VMEM_SHARED((N_PAD, dh), jnp.float32),
            pltpu.SemaphoreType.DMA,
        ],
    )
    def k(lo_hbm, hi_hbm, src_hbm, dst_hbm, olo_hbm, ohi_hbm,
          srcv, dstv, rows, zbuf, acc_sh, sem):
        c = lax.axis_index("c")
        s = lax.axis_index("s")

        def fill_zero(r, _):
            def col(j, _):
                zbuf[r, pl.ds(j * 16, 16)] = jnp.zeros((16,), jnp.float32)
                return 0
            return lax.fori_loop(0, dh // 16, col, 0)

        lax.fori_loop(0, ZR, fill_zero, 0)

        def zero_blk(j, _):
            pltpu.sync_copy(zbuf, acc_sh.at[pl.ds(s * TPN + j * ZR, ZR)])
            return 0

        lax.fori_loop(0, TPN // ZR, zero_blk, 0)
        pltpu.sync_copy(src_hbm.at[pl.ds(s * MP_NB, MP_NB)], srcv)
        pltpu.sync_copy(dst_hbm.at[pl.ds(s * MP_NB, MP_NB)], dstv)
        plsc.subcore_barrier()

        def body(hp_hbm, out_hbm):
            def mp(b, _):
                pltpu.async_copy(hp_hbm.at[srcv.at[b]], rows, sem).wait()
                pltpu.sync_copy(rows, acc_sh.at[dstv.at[b]], add=True)
                return 0

            lax.fori_loop(0, MP_NB, mp, 0)
            plsc.subcore_barrier()

            def wout(j, _):
                pltpu.sync_copy(acc_sh.at[pl.ds(s * TPN + j * ZR, ZR)], zbuf)
                pltpu.sync_copy(zbuf, out_hbm.at[pl.ds(s * TPN + j * ZR, ZR)])
                return 0

            lax.fori_loop(0, TPN // ZR, wout, 0)

        @pl.when(c == 0)
        def _():
            body(lo_hbm, olo_hbm)

        @pl.when(c == 1)
        def _():
            body(hi_hbm, ohi_hbm)

    return k(hp_lo, hp_hi, src2d, dst2d)


def _sc_msgpass_edges(hp, src2d, dst2d):
    dh = 128
    wnb = MP_NB // NC

    @functools.partial(
        pl.kernel,
        out_type=jax.ShapeDtypeStruct((NC * N_PAD, dh), jnp.float32),
        mesh=_mesh,
        scratch_types=[
            pltpu.VMEM((wnb, MP_K), jnp.int32),
            pltpu.VMEM((wnb, MP_K), jnp.int32),
            pltpu.VMEM((MP_K, dh), jnp.float32),
            pltpu.VMEM((ZR, dh), jnp.float32),
            pltpu.VMEM_SHARED((N_PAD, dh), jnp.float32),
            pltpu.SemaphoreType.DMA,
        ],
    )
    def k(hp_hbm, src_hbm, dst_hbm, out_hbm,
          srcv, dstv, rows, zbuf, acc_sh, sem):
        c = lax.axis_index("c")
        s = lax.axis_index("s")
        w = c * NS + s

        def fill_zero(r, _):
            def col(j, _):
                zbuf[r, pl.ds(j * 16, 16)] = jnp.zeros((16,), jnp.float32)
                return 0
            return lax.fori_loop(0, dh // 16, col, 0)

        lax.fori_loop(0, ZR, fill_zero, 0)

        def zero_blk(j, _):
            pltpu.sync_copy(zbuf, acc_sh.at[pl.ds(s * TPN + j * ZR, ZR)])
            return 0

        lax.fori_loop(0, TPN // ZR, zero_blk, 0)
        pltpu.sync_copy(src_hbm.at[pl.ds(w * wnb, wnb)], srcv)
        pltpu.sync_copy(dst_hbm.at[pl.ds(w * wnb, wnb)], dstv)
        plsc.subcore_barrier()

        def mp(b, _):
            pltpu.async_copy(hp_hbm.at[srcv.at[b]], rows, sem).wait()
            pltpu.sync_copy(rows, acc_sh.at[dstv.at[b]], add=True)
            return 0

        lax.fori_loop(0, wnb, mp, 0)
        plsc.subcore_barrier()

        def wout(j, _):
            pltpu.sync_copy(acc_sh.at[pl.ds(s * TPN + j * ZR, ZR)], zbuf)
            base = pl.multiple_of(c * N_PAD + s * TPN + j * ZR, ZR)
            pltpu.sync_copy(zbuf, out_hbm.at[pl.ds(base, ZR)])
            return 0

        lax.fori_loop(0, TPN // ZR, wout, 0)

    return k(hp, src2d, dst2d)


_R = 1024
_G = N_PAD // _R


def _dinv_block(degp_ref):
    deg = jnp.sum(degp_ref[...], axis=1, keepdims=True) + 1.0
    return lax.rsqrt(deg)


def _tc1_body(x_ref, w_ref, degp_ref, lo_ref, hi_ref):
    dinv = _dinv_block(degp_ref)
    h = jnp.dot(x_ref[...], w_ref[...], preferred_element_type=jnp.float32)
    hp = h * dinv
    lo_ref[...] = hp[:, : D_HID // 2]
    hi_ref[...] = hp[:, D_HID // 2:]


def _tc2_body(alo_ref, ahi_ref, plo_ref, phi_ref, degp_ref, b1_ref, w2_ref,
              o_ref):
    dinv = _dinv_block(degp_ref)
    zlo = jnp.maximum(dinv * (alo_ref[...] + plo_ref[...])
                      + b1_ref[0:1, : D_HID // 2], 0.0)
    zhi = jnp.maximum(dinv * (ahi_ref[...] + phi_ref[...])
                      + b1_ref[0:1, D_HID // 2:], 0.0)
    z = jnp.concatenate([zlo, zhi], axis=1)
    h2 = jnp.dot(z, w2_ref[...], preferred_element_type=jnp.float32)
    h2p = h2 * dinv
    o_ref[...] = jnp.concatenate(
        [h2p, jnp.zeros((h2p.shape[0], 128 - D_OUT), jnp.float32)], axis=1)


def _tc3_body(a0_ref, a1_ref, p_ref, degp_ref, b2_ref, out_ref):
    dinv = _dinv_block(degp_ref)
    acc = a0_ref[:, :D_OUT] + a1_ref[:, :D_OUT] + p_ref[:, :D_OUT]
    out_ref[...] = dinv * acc + b2_ref[0:1, :]


def _rows_spec(cols):
    return pl.BlockSpec((_R, cols), lambda i: (i, 0))


def _full_spec(r, c):
    return pl.BlockSpec((r, c), lambda i: (0, 0))


def kernel(x, edge_index, W1, b1, W2, b2):
    src = edge_index[0].astype(jnp.int32)
    dst = edge_index[1].astype(jnp.int32)
    src_mp = src.reshape(E // MP_K, MP_K)
    dst_mp = dst.reshape(E // MP_K, MP_K)
    xp = jnp.pad(x, ((0, N_PAD - N), (0, 0)))
    b1r = b1.reshape(1, D_HID)
    b2r = b2.reshape(1, D_OUT)

    deg_hist = _sc_degree(dst_mp)
    degp = deg_hist.reshape(NC, N_PAD, DEG_W)[:, :, 0].T

    h1p_lo, h1p_hi = pl.pallas_call(
        _tc1_body,
        grid=(_G,),
        in_specs=[
            _rows_spec(D_IN),
            _full_spec(D_IN, D_HID),
            _rows_spec(NC),
        ],
        out_specs=[_rows_spec(D_HID // 2)] * 2,
        out_shape=[jax.ShapeDtypeStruct((N_PAD, D_HID // 2), jnp.float32)] * 2,
    )(xp, W1, degp)

    acc1_lo, acc1_hi = _sc_msgpass(h1p_lo, h1p_hi, src_mp, dst_mp, D_HID // 2)

    h2p = pl.pallas_call(
        _tc2_body,
        grid=(_G,),
        in_specs=[
            _rows_spec(D_HID // 2), _rows_spec(D_HID // 2),
            _rows_spec(D_HID // 2), _rows_spec(D_HID // 2),
            _rows_spec(NC),
            _full_spec(1, D_HID),
            _full_spec(D_HID, D_OUT),
        ],
        out_specs=_rows_spec(128),
        out_shape=jax.ShapeDtypeStruct((N_PAD, 128), jnp.float32),
    )(acc1_lo, acc1_hi, h1p_lo, h1p_hi, degp, b1r, W2)

    acc2 = _sc_msgpass_edges(h2p, src_mp, dst_mp).reshape(NC, N_PAD, 128)
    acc2_p0, acc2_p1 = acc2[0], acc2[1]

    out = pl.pallas_call(
        _tc3_body,
        grid=(_G,),
        in_specs=[
            _rows_spec(128), _rows_spec(128), _rows_spec(128),
            _rows_spec(NC),
            _full_spec(1, D_OUT),
        ],
        out_specs=_rows_spec(D_OUT),
        out_shape=jax.ShapeDtypeStruct((N_PAD, D_OUT), jnp.float32),
    )(acc2_p0, acc2_p1, h2p, degp, b2r)

    return out[:N]


# speedup vs baseline: 15.3117x; 1.2863x over previous
import functools

import jax
import jax.numpy as jnp
from jax import lax
from jax.experimental import pallas as pl
from jax.experimental.pallas import tpu as pltpu
from jax.experimental.pallas import tpu_sc as plsc

N = 10000
N_PAD = 10240
E = 160000
D_IN = 256
D_HID = 256
D_OUT = 64

NC = 2
NS = 16
TPN = N_PAD // NS
ZR = 64

MP_K = 125
MP_NB = (E // NS) // MP_K

DEG_W = 128

_mesh = plsc.VectorSubcoreMesh(core_axis_name="c", subcore_axis_name="s")


def _sc_degree(dst2d):
    wnb = MP_NB // NC

    @functools.partial(
        pl.kernel,
        out_type=jax.ShapeDtypeStruct((NC * N_PAD, DEG_W), jnp.float32),
        mesh=_mesh,
        scratch_types=[
            pltpu.VMEM((wnb, MP_K), jnp.int32),
            pltpu.VMEM((MP_K, DEG_W), jnp.float32),
            pltpu.VMEM((ZR, DEG_W), jnp.float32),
            pltpu.VMEM_SHARED((N_PAD, DEG_W), jnp.float32),
        ],
    )
    def k(dst_hbm, out_hbm, dstv, ones, zbuf, hist_sh):
        c = lax.axis_index("c")
        s = lax.axis_index("s")
        w = c * NS + s

        def fill_ones(r, _):
            def col(j, _):
                ones[r, pl.ds(j * 16, 16)] = jnp.ones((16,), jnp.float32)
                return 0
            return lax.fori_loop(0, DEG_W // 16, col, 0)

        lax.fori_loop(0, MP_K, fill_ones, 0)

        def fill_zero(r, _):
            def col(j, _):
                zbuf[r, pl.ds(j * 16, 16)] = jnp.zeros((16,), jnp.float32)
                return 0
            return lax.fori_loop(0, DEG_W // 16, col, 0)

        lax.fori_loop(0, ZR, fill_zero, 0)

        def zero_blk(j, _):
            pltpu.sync_copy(zbuf, hist_sh.at[pl.ds(s * TPN + j * ZR, ZR)])
            return 0

        lax.fori_loop(0, TPN // ZR, zero_blk, 0)
        pltpu.sync_copy(dst_hbm.at[pl.ds(w * wnb, wnb)], dstv)
        plsc.subcore_barrier()

        def scat(b, _):
            pltpu.sync_copy(ones, hist_sh.at[dstv.at[b]], add=True)
            return 0

        lax.fori_loop(0, wnb, scat, 0)
        plsc.subcore_barrier()

        def wout(j, _):
            pltpu.sync_copy(hist_sh.at[pl.ds(s * TPN + j * ZR, ZR)], zbuf)
            base = pl.multiple_of(c * N_PAD + s * TPN + j * ZR, ZR)
            pltpu.sync_copy(zbuf, out_hbm.at[pl.ds(base, ZR)])
            return 0

        lax.fori_loop(0, TPN // ZR, wout, 0)

    return k(dst2d)


ICH = 8


def _ring_row(b):
    return ((b // ICH) % 2) * ICH + (b % ICH)


def _mp_pipeline(hp_hbm, acc_sh, srcr, dstr, rows0, rows1,
                 sem0, sem1, semi_s, semi_d, src_hbm, dst_hbm,
                 base_row, wrows):
    pltpu.sync_copy(src_hbm.at[pl.ds(base_row, ICH)], srcr.at[pl.ds(0, ICH)])
    pltpu.sync_copy(dst_hbm.at[pl.ds(base_row, ICH)], dstr.at[pl.ds(0, ICH)])
    pltpu.async_copy(hp_hbm.at[srcr.at[0]], rows0, sem0)

    def pair(p, _):
        b0 = 2 * p
        b1 = b0 + 1

        @pl.when(jnp.logical_and(b0 % ICH == 0, b0 + ICH < wrows))
        def _():
            cn = b0 // ICH + 1
            off = pl.multiple_of((cn % 2) * ICH, ICH)
            pltpu.async_copy(src_hbm.at[pl.ds(base_row + cn * ICH, ICH)],
                             srcr.at[pl.ds(off, ICH)], semi_s)
            pltpu.async_copy(dst_hbm.at[pl.ds(base_row + cn * ICH, ICH)],
                             dstr.at[pl.ds(off, ICH)], semi_d)

        rr0 = _ring_row(b0)
        rr1 = _ring_row(b1)
        pltpu.async_copy(hp_hbm.at[srcr.at[rr1]], rows1, sem1)
        pltpu.make_async_copy(hp_hbm.at[srcr.at[rr0]], rows0, sem0).wait()
        pltpu.sync_copy(rows0, acc_sh.at[dstr.at[rr0]], add=True)

        @pl.when(b0 + 2 < wrows)
        def _():
            @pl.when((b0 + 2) % ICH == 0)
            def _():
                pltpu.make_async_copy(
                    src_hbm.at[pl.ds(base_row, ICH)],
                    srcr.at[pl.ds(0, ICH)], semi_s).wait()
                pltpu.make_async_copy(
                    dst_hbm.at[pl.ds(base_row, ICH)],
                    dstr.at[pl.ds(0, ICH)], semi_d).wait()

            pltpu.async_copy(hp_hbm.at[srcr.at[_ring_row(b0 + 2)]],
                             rows0, sem0)

        pltpu.make_async_copy(hp_hbm.at[srcr.at[rr1]], rows1, sem1).wait()
        pltpu.sync_copy(rows1, acc_sh.at[dstr.at[rr1]], add=True)
        return 0

    lax.fori_loop(0, wrows // 2, pair, 0)


def _zero_fill(zbuf, dh):
    def fill_zero(r, _):
        def col(j, _):
            zbuf[r, pl.ds(j * 16, 16)] = jnp.zeros((16,), jnp.float32)
            return 0
        return lax.fori_loop(0, dh // 16, col, 0)

    lax.fori_loop(0, ZR, fill_zero, 0)


def _sc_msgpass(hp_lo, hp_hi, src2d, dst2d, dh):
    @functools.partial(
        pl.kernel,
        out_type=[jax.ShapeDtypeStruct((N_PAD, dh), jnp.float32)] * 2,
        mesh=_mesh,
        scratch_types=[
            pltpu.VMEM((2 * ICH, MP_K), jnp.int32),
            pltpu.VMEM((2 * ICH, MP_K), jnp.int32),
            pltpu.VMEM((MP_K, dh), jnp.float32),
            pltpu.VMEM((MP_K, dh), jnp.float32),
            pltpu.VMEM((ZR, dh), jnp.float32),
            pltpu.VMEM_SHARED((N_PAD, dh), jnp.float32),
            pltpu.SemaphoreType.DMA,
            pltpu.SemaphoreType.DMA,
            pltpu.SemaphoreType.DMA,
            pltpu.SemaphoreType.DMA,
        ],
    )
    def k(lo_hbm, hi_hbm, src_hbm, dst_hbm, olo_hbm, ohi_hbm,
          srcr, dstr, rows0, rows1, zbuf, acc_sh, sem0, sem1, semi_s, semi_d):
        c = lax.axis_index("c")
        s = lax.axis_index("s")

        _zero_fill(zbuf, dh)

        def zero_blk(j, _):
            pltpu.sync_copy(zbuf, acc_sh.at[pl.ds(s * TPN + j * ZR, ZR)])
            return 0

        lax.fori_loop(0, TPN // ZR, zero_blk, 0)
        plsc.subcore_barrier()

        def body(hp_hbm, out_hbm):
            _mp_pipeline(hp_hbm, acc_sh, srcr, dstr, rows0, rows1,
                         sem0, sem1, semi_s, semi_d, src_hbm, dst_hbm,
                         s * MP_NB, MP_NB)
            plsc.subcore_barrier()

            def wout(j, _):
                pltpu.sync_copy(acc_sh.at[pl.ds(s * TPN + j * ZR, ZR)], zbuf)
                pltpu.sync_copy(zbuf, out_hbm.at[pl.ds(s * TPN + j * ZR, ZR)])
                return 0

            lax.fori_loop(0, TPN // ZR, wout, 0)

        @pl.when(c == 0)
        def _():
            body(lo_hbm, olo_hbm)

        @pl.when(c == 1)
        def _():
            body(hi_hbm, ohi_hbm)

    return k(hp_lo, hp_hi, src2d, dst2d)


def _sc_msgpass_edges(hp, src2d, dst2d):
    dh = 128
    wnb = MP_NB // NC

    @functools.partial(
        pl.kernel,
        out_type=jax.ShapeDtypeStruct((NC * N_PAD, dh), jnp.float32),
        mesh=_mesh,
        scratch_types=[
            pltpu.VMEM((2 * ICH, MP_K), jnp.int32),
            pltpu.VMEM((2 * ICH, MP_K), jnp.int32),
            pltpu.VMEM((MP_K, dh), jnp.float32),
            pltpu.VMEM((MP_K, dh), jnp.float32),
            pltpu.VMEM((ZR, dh), jnp.float32),
            pltpu.VMEM_SHARED((N_PAD, dh), jnp.float32),
            pltpu.SemaphoreType.DMA,
            pltpu.SemaphoreType.DMA,
            pltpu.SemaphoreType.DMA,
            pltpu.SemaphoreType.DMA,
        ],
    )
    def k(hp_hbm, src_hbm, dst_hbm, out_hbm,
          srcr, dstr, rows0, rows1, zbuf, acc_sh,
          sem0, sem1, semi_s, semi_d):
        c = lax.axis_index("c")
        s = lax.axis_index("s")
        w = c * NS + s

        _zero_fill(zbuf, dh)

        def zero_blk(j, _):
            pltpu.sync_copy(zbuf, acc_sh.at[pl.ds(s * TPN + j * ZR, ZR)])
            return 0

        lax.fori_loop(0, TPN // ZR, zero_blk, 0)
        plsc.subcore_barrier()

        _mp_pipeline(hp_hbm, acc_sh, srcr, dstr, rows0, rows1,
                     sem0, sem1, semi_s, semi_d, src_hbm, dst_hbm,
                     w * wnb, wnb)
        plsc.subcore_barrier()

        def wout(j, _):
            pltpu.sync_copy(acc_sh.at[pl.ds(s * TPN + j * ZR, ZR)], zbuf)
            base = pl.multiple_of(c * N_PAD + s * TPN + j * ZR, ZR)
            pltpu.sync_copy(zbuf, out_hbm.at[pl.ds(base, ZR)])
            return 0

        lax.fori_loop(0, TPN // ZR, wout, 0)

    return k(hp, src2d, dst2d)


_R = 1024
_G = N_PAD // _R


def _dinv_block(degp_ref):
    deg = jnp.sum(degp_ref[...], axis=1, keepdims=True) + 1.0
    return lax.rsqrt(deg)


def _tc1_body(x_ref, w_ref, degp_ref, lo_ref, hi_ref):
    dinv = _dinv_block(degp_ref)
    h = jnp.dot(x_ref[...], w_ref[...], preferred_element_type=jnp.float32)
    hp = h * dinv
    lo_ref[...] = hp[:, : D_HID // 2]
    hi_ref[...] = hp[:, D_HID // 2:]


def _tc2_body(alo_ref, ahi_ref, plo_ref, phi_ref, degp_ref, b1_ref, w2_ref,
              o_ref):
    dinv = _dinv_block(degp_ref)
    zlo = jnp.maximum(dinv * (alo_ref[...] + plo_ref[...])
                      + b1_ref[0:1, : D_HID // 2], 0.0)
    zhi = jnp.maximum(dinv * (ahi_ref[...] + phi_ref[...])
                      + b1_ref[0:1, D_HID // 2:], 0.0)
    z = jnp.concatenate([zlo, zhi], axis=1)
    h2 = jnp.dot(z, w2_ref[...], preferred_element_type=jnp.float32)
    h2p = h2 * dinv
    o_ref[...] = jnp.concatenate(
        [h2p, jnp.zeros((h2p.shape[0], 128 - D_OUT), jnp.float32)], axis=1)


def _tc3_body(a0_ref, a1_ref, p_ref, degp_ref, b2_ref, out_ref):
    dinv = _dinv_block(degp_ref)
    acc = a0_ref[:, :D_OUT] + a1_ref[:, :D_OUT] + p_ref[:, :D_OUT]
    out_ref[...] = dinv * acc + b2_ref[0:1, :]


def _rows_spec(cols):
    return pl.BlockSpec((_R, cols), lambda i: (i, 0))


def _full_spec(r, c):
    return pl.BlockSpec((r, c), lambda i: (0, 0))


def kernel(x, edge_index, W1, b1, W2, b2):
    src = edge_index[0].astype(jnp.int32)
    dst = edge_index[1].astype(jnp.int32)
    src_mp = src.reshape(E // MP_K, MP_K)
    dst_mp = dst.reshape(E // MP_K, MP_K)
    xp = jnp.pad(x, ((0, N_PAD - N), (0, 0)))
    b1r = b1.reshape(1, D_HID)
    b2r = b2.reshape(1, D_OUT)

    deg_hist = _sc_degree(dst_mp)
    degp = deg_hist.reshape(NC, N_PAD, DEG_W)[:, :, 0].T

    h1p_lo, h1p_hi = pl.pallas_call(
        _tc1_body,
        grid=(_G,),
        in_specs=[
            _rows_spec(D_IN),
            _full_spec(D_IN, D_HID),
            _rows_spec(NC),
        ],
        out_specs=[_rows_spec(D_HID // 2)] * 2,
        out_shape=[jax.ShapeDtypeStruct((N_PAD, D_HID // 2), jnp.float32)] * 2,
    )(xp, W1, degp)

    acc1_lo, acc1_hi = _sc_msgpass(h1p_lo, h1p_hi, src_mp, dst_mp, D_HID // 2)

    h2p = pl.pallas_call(
        _tc2_body,
        grid=(_G,),
        in_specs=[
            _rows_spec(D_HID // 2), _rows_spec(D_HID // 2),
            _rows_spec(D_HID // 2), _rows_spec(D_HID // 2),
            _rows_spec(NC),
            _full_spec(1, D_HID),
            _full_spec(D_HID, D_OUT),
        ],
        out_specs=_rows_spec(128),
        out_shape=jax.ShapeDtypeStruct((N_PAD, 128), jnp.float32),
    )(acc1_lo, acc1_hi, h1p_lo, h1p_hi, degp, b1r, W2)

    acc2 = _sc_msgpass_edges(h2p, src_mp, dst_mp).reshape(NC, N_PAD, 128)
    acc2_p0, acc2_p1 = acc2[0], acc2[1]

    out = pl.pallas_call(
        _tc3_body,
        grid=(_G,),
        in_specs=[
            _rows_spec(128), _rows_spec(128), _rows_spec(128),
            _rows_spec(NC),
            _full_spec(1, D_OUT),
        ],
        out_specs=_rows_spec(D_OUT),
        out_shape=jax.ShapeDtypeStruct((N_PAD, D_OUT), jnp.float32),
    )(acc2_p0, acc2_p1, h2p, degp, b2r)

    return out[:N]


# speedup vs baseline: 18.4720x; 1.2064x over previous
import functools

import jax
import jax.numpy as jnp
from jax import lax
from jax.experimental import pallas as pl
from jax.experimental.pallas import tpu as pltpu
from jax.experimental.pallas import tpu_sc as plsc

N = 10000
N_PAD = 10240
E = 160000
D_IN = 256
D_HID = 256
D_OUT = 64

NC = 2
NS = 16
TPN = N_PAD // NS
ZR = 64

MP_K = 125
MP_NB = (E // NS) // MP_K

DEG_W = 128

_mesh = plsc.VectorSubcoreMesh(core_axis_name="c", subcore_axis_name="s")


def _sc_degree(dst2d):
    wnb = MP_NB // NC

    @functools.partial(
        pl.kernel,
        out_type=jax.ShapeDtypeStruct((NC * N_PAD, DEG_W), jnp.float32),
        mesh=_mesh,
        scratch_types=[
            pltpu.VMEM((wnb, MP_K), jnp.int32),
            pltpu.VMEM((MP_K, DEG_W), jnp.float32),
            pltpu.VMEM((ZR, DEG_W), jnp.float32),
            pltpu.VMEM_SHARED((N_PAD, DEG_W), jnp.float32),
        ],
    )
    def k(dst_hbm, out_hbm, dstv, ones, zbuf, hist_sh):
        c = lax.axis_index("c")
        s = lax.axis_index("s")
        w = c * NS + s

        def fill_ones(r, _):
            def col(j, _):
                ones[r, pl.ds(j * 16, 16)] = jnp.ones((16,), jnp.float32)
                return 0
            return lax.fori_loop(0, DEG_W // 16, col, 0)

        lax.fori_loop(0, MP_K, fill_ones, 0)

        def fill_zero(r, _):
            def col(j, _):
                zbuf[r, pl.ds(j * 16, 16)] = jnp.zeros((16,), jnp.float32)
                return 0
            return lax.fori_loop(0, DEG_W // 16, col, 0)

        lax.fori_loop(0, ZR, fill_zero, 0)

        def zero_blk(j, _):
            pltpu.sync_copy(zbuf, hist_sh.at[pl.ds(s * TPN + j * ZR, ZR)])
            return 0

        lax.fori_loop(0, TPN // ZR, zero_blk, 0)
        pltpu.sync_copy(dst_hbm.at[pl.ds(w * wnb, wnb)], dstv)
        plsc.subcore_barrier()

        def scat(b, _):
            pltpu.sync_copy(ones, hist_sh.at[dstv.at[b]], add=True)
            return 0

        lax.fori_loop(0, wnb, scat, 0)
        plsc.subcore_barrier()

        def wout(j, _):
            pltpu.sync_copy(hist_sh.at[pl.ds(s * TPN + j * ZR, ZR)], zbuf)
            base = pl.multiple_of(c * N_PAD + s * TPN + j * ZR, ZR)
            pltpu.sync_copy(zbuf, out_hbm.at[pl.ds(base, ZR)])
            return 0

        lax.fori_loop(0, TPN // ZR, wout, 0)

    return k(dst2d)


ICH = 8


def _ring_row(b):
    return ((b // ICH) % 2) * ICH + (b % ICH)


def _mp_pipeline(hp_hbm, acc_sh, srcr, dstr, rows0, rows1,
                 sem0, sem1, semi_s, semi_d, src_hbm, dst_hbm,
                 base_row, wrows):
    pltpu.sync_copy(src_hbm.at[pl.ds(base_row, ICH)], srcr.at[pl.ds(0, ICH)])
    pltpu.sync_copy(dst_hbm.at[pl.ds(base_row, ICH)], dstr.at[pl.ds(0, ICH)])
    pltpu.async_copy(hp_hbm.at[srcr.at[0]], rows0, sem0)

    def pair(p, _):
        b0 = 2 * p
        b1 = b0 + 1

        @pl.when(jnp.logical_and(b0 % ICH == 0, b0 + ICH < wrows))
        def _():
            cn = b0 // ICH + 1
            off = pl.multiple_of((cn % 2) * ICH, ICH)
            pltpu.async_copy(src_hbm.at[pl.ds(base_row + cn * ICH, ICH)],
                             srcr.at[pl.ds(off, ICH)], semi_s)
            pltpu.async_copy(dst_hbm.at[pl.ds(base_row + cn * ICH, ICH)],
                             dstr.at[pl.ds(off, ICH)], semi_d)

        rr0 = _ring_row(b0)
        rr1 = _ring_row(b1)
        pltpu.async_copy(hp_hbm.at[srcr.at[rr1]], rows1, sem1)
        pltpu.make_async_copy(hp_hbm.at[srcr.at[rr0]], rows0, sem0).wait()
        pltpu.sync_copy(rows0, acc_sh.at[dstr.at[rr0]], add=True)

        @pl.when(b0 + 2 < wrows)
        def _():
            @pl.when((b0 + 2) % ICH == 0)
            def _():
                pltpu.make_async_copy(
                    src_hbm.at[pl.ds(base_row, ICH)],
                    srcr.at[pl.ds(0, ICH)], semi_s).wait()
                pltpu.make_async_copy(
                    dst_hbm.at[pl.ds(base_row, ICH)],
                    dstr.at[pl.ds(0, ICH)], semi_d).wait()

            pltpu.async_copy(hp_hbm.at[srcr.at[_ring_row(b0 + 2)]],
                             rows0, sem0)

        pltpu.make_async_copy(hp_hbm.at[srcr.at[rr1]], rows1, sem1).wait()
        pltpu.sync_copy(rows1, acc_sh.at[dstr.at[rr1]], add=True)
        return 0

    lax.fori_loop(0, wrows // 2, pair, 0)


def _zero_fill(zbuf, dh):
    def fill_zero(r, _):
        def col(j, _):
            zbuf[r, pl.ds(j * 16, 16)] = jnp.zeros((16,), jnp.float32)
            return 0
        return lax.fori_loop(0, dh // 16, col, 0)

    lax.fori_loop(0, ZR, fill_zero, 0)


def _sc_msgpass(hp_lo, hp_hi, src2d, dst2d, dh):
    @functools.partial(
        pl.kernel,
        out_type=[jax.ShapeDtypeStruct((N_PAD, dh), jnp.float32)] * 2,
        mesh=_mesh,
        scratch_types=[
            pltpu.VMEM((2 * ICH, MP_K), jnp.int32),
            pltpu.VMEM((2 * ICH, MP_K), jnp.int32),
            pltpu.VMEM((MP_K, dh), jnp.float32),
            pltpu.VMEM((MP_K, dh), jnp.float32),
            pltpu.VMEM((ZR, dh), jnp.float32),
            pltpu.VMEM_SHARED((N_PAD, dh), jnp.float32),
            pltpu.SemaphoreType.DMA,
            pltpu.SemaphoreType.DMA,
            pltpu.SemaphoreType.DMA,
            pltpu.SemaphoreType.DMA,
        ],
    )
    def k(lo_hbm, hi_hbm, src_hbm, dst_hbm, olo_hbm, ohi_hbm,
          srcr, dstr, rows0, rows1, zbuf, acc_sh, sem0, sem1, semi_s, semi_d):
        c = lax.axis_index("c")
        s = lax.axis_index("s")

        _zero_fill(zbuf, dh)

        def zero_blk(j, _):
            pltpu.sync_copy(zbuf, acc_sh.at[pl.ds(s * TPN + j * ZR, ZR)])
            return 0

        lax.fori_loop(0, TPN // ZR, zero_blk, 0)
        plsc.subcore_barrier()

        def body(hp_hbm, out_hbm):
            _mp_pipeline(hp_hbm, acc_sh, srcr, dstr, rows0, rows1,
                         sem0, sem1, semi_s, semi_d, src_hbm, dst_hbm,
                         s * MP_NB, MP_NB)
            plsc.subcore_barrier()

            def wout(j, _):
                pltpu.sync_copy(acc_sh.at[pl.ds(s * TPN + j * ZR, ZR)], zbuf)
                pltpu.sync_copy(zbuf, out_hbm.at[pl.ds(s * TPN + j * ZR, ZR)])
                return 0

            lax.fori_loop(0, TPN // ZR, wout, 0)

        @pl.when(c == 0)
        def _():
            body(lo_hbm, olo_hbm)

        @pl.when(c == 1)
        def _():
            body(hi_hbm, ohi_hbm)

    return k(hp_lo, hp_hi, src2d, dst2d)


def _sc_msgpass_edges(hp, src2d, dst2d):
    dh = 128
    wnb = MP_NB // NC

    @functools.partial(
        pl.kernel,
        out_type=jax.ShapeDtypeStruct((NC * N_PAD, dh), jnp.float32),
        mesh=_mesh,
        scratch_types=[
            pltpu.VMEM((2 * ICH, MP_K), jnp.int32),
            pltpu.VMEM((2 * ICH, MP_K), jnp.int32),
            pltpu.VMEM((MP_K, dh), jnp.float32),
            pltpu.VMEM((MP_K, dh), jnp.float32),
            pltpu.VMEM((ZR, dh), jnp.float32),
            pltpu.VMEM_SHARED((N_PAD, dh), jnp.float32),
            pltpu.SemaphoreType.DMA,
            pltpu.SemaphoreType.DMA,
            pltpu.SemaphoreType.DMA,
            pltpu.SemaphoreType.DMA,
        ],
    )
    def k(hp_hbm, src_hbm, dst_hbm, out_hbm,
          srcr, dstr, rows0, rows1, zbuf, acc_sh,
          sem0, sem1, semi_s, semi_d):
        c = lax.axis_index("c")
        s = lax.axis_index("s")
        w = c * NS + s

        _zero_fill(zbuf, dh)

        def zero_blk(j, _):
            pltpu.sync_copy(zbuf, acc_sh.at[pl.ds(s * TPN + j * ZR, ZR)])
            return 0

        lax.fori_loop(0, TPN // ZR, zero_blk, 0)
        plsc.subcore_barrier()

        _mp_pipeline(hp_hbm, acc_sh, srcr, dstr, rows0, rows1,
                     sem0, sem1, semi_s, semi_d, src_hbm, dst_hbm,
                     w * wnb, wnb)
        plsc.subcore_barrier()

        def wout(j, _):
            pltpu.sync_copy(acc_sh.at[pl.ds(s * TPN + j * ZR, ZR)], zbuf)
            base = pl.multiple_of(c * N_PAD + s * TPN + j * ZR, ZR)
            pltpu.sync_copy(zbuf, out_hbm.at[pl.ds(base, ZR)])
            return 0

        lax.fori_loop(0, TPN // ZR, wout, 0)

    return k(hp, src2d, dst2d)


_R = 1024
_G = N_PAD // _R


def _dinv_block(dega_ref, degb_ref):
    deg = dega_ref[:, 0:1] + degb_ref[:, 0:1] + 1.0
    return lax.rsqrt(deg)


def _tc1_body(x_ref, w_ref, dega_ref, degb_ref, lo_ref, hi_ref):
    dinv = _dinv_block(dega_ref, degb_ref)
    h = jnp.dot(x_ref[...], w_ref[...], preferred_element_type=jnp.float32)
    hp = h * dinv
    lo_ref[...] = hp[:, : D_HID // 2]
    hi_ref[...] = hp[:, D_HID // 2:]


def _tc2_body(alo_ref, ahi_ref, plo_ref, phi_ref, dega_ref, degb_ref,
              b1_ref, w2_ref, o_ref):
    dinv = _dinv_block(dega_ref, degb_ref)
    zlo = jnp.maximum(dinv * (alo_ref[...] + plo_ref[...])
                      + b1_ref[0:1, : D_HID // 2], 0.0)
    zhi = jnp.maximum(dinv * (ahi_ref[...] + phi_ref[...])
                      + b1_ref[0:1, D_HID // 2:], 0.0)
    z = jnp.concatenate([zlo, zhi], axis=1)
    h2 = jnp.dot(z, w2_ref[...], preferred_element_type=jnp.float32)
    h2p = h2 * dinv
    o_ref[...] = jnp.concatenate(
        [h2p, jnp.zeros((h2p.shape[0], 128 - D_OUT), jnp.float32)], axis=1)


def _tc3_body(a0_ref, a1_ref, p_ref, dega_ref, degb_ref, b2_ref, out_ref):
    dinv = _dinv_block(dega_ref, degb_ref)
    acc = a0_ref[:, :D_OUT] + a1_ref[:, :D_OUT] + p_ref[:, :D_OUT]
    out_ref[...] = dinv * acc + b2_ref[0:1, :]


def _rows_spec(cols):
    return pl.BlockSpec((_R, cols), lambda i: (i, 0))


_DEG_SPECS = [pl.BlockSpec((_R, DEG_W), lambda i: (i, 0)),
              pl.BlockSpec((_R, DEG_W), lambda i: (i + _G, 0))]


def _full_spec(r, c):
    return pl.BlockSpec((r, c), lambda i: (0, 0))


def kernel(x, edge_index, W1, b1, W2, b2):
    src = edge_index[0].astype(jnp.int32)
    dst = edge_index[1].astype(jnp.int32)
    src_mp = src.reshape(E // MP_K, MP_K)
    dst_mp = dst.reshape(E // MP_K, MP_K)
    xp = jnp.pad(x, ((0, N_PAD - N), (0, 0)))
    b1r = b1.reshape(1, D_HID)
    b2r = b2.reshape(1, D_OUT)

    deg_hist = _sc_degree(dst_mp)

    h1p_lo, h1p_hi = pl.pallas_call(
        _tc1_body,
        grid=(_G,),
        in_specs=[
            _rows_spec(D_IN),
            _full_spec(D_IN, D_HID),
        ] + _DEG_SPECS,
        out_specs=[_rows_spec(D_HID // 2)] * 2,
        out_shape=[jax.ShapeDtypeStruct((N_PAD, D_HID // 2), jnp.float32)] * 2,
    )(xp, W1, deg_hist, deg_hist)

    acc1_lo, acc1_hi = _sc_msgpass(h1p_lo, h1p_hi, src_mp, dst_mp, D_HID // 2)

    h2p = pl.pallas_call(
        _tc2_body,
        grid=(_G,),
        in_specs=[
            _rows_spec(D_HID // 2), _rows_spec(D_HID // 2),
            _rows_spec(D_HID // 2), _rows_spec(D_HID // 2),
        ] + _DEG_SPECS + [
            _full_spec(1, D_HID),
            _full_spec(D_HID, D_OUT),
        ],
        out_specs=_rows_spec(128),
        out_shape=jax.ShapeDtypeStruct((N_PAD, 128), jnp.float32),
    )(acc1_lo, acc1_hi, h1p_lo, h1p_hi, deg_hist, deg_hist, b1r, W2)

    acc2 = _sc_msgpass_edges(h2p, src_mp, dst_mp).reshape(NC, N_PAD, 128)
    acc2_p0, acc2_p1 = acc2[0], acc2[1]

    out = pl.pallas_call(
        _tc3_body,
        grid=(_G,),
        in_specs=[
            _rows_spec(128), _rows_spec(128), _rows_spec(128),
        ] + _DEG_SPECS + [
            _full_spec(1, D_OUT),
        ],
        out_specs=_rows_spec(D_OUT),
        out_shape=jax.ShapeDtypeStruct((N_PAD, D_OUT), jnp.float32),
    )(acc2_p0, acc2_p1, h2p, deg_hist, deg_hist, b2r)

    return out[:N]


# speedup vs baseline: 19.1698x; 1.0378x over previous
import functools

import jax
import jax.numpy as jnp
from jax import lax
from jax.experimental import pallas as pl
from jax.experimental.pallas import tpu as pltpu
from jax.experimental.pallas import tpu_sc as plsc

N = 10000
N_PAD = 10240
E = 160000
D_IN = 256
D_HID = 256
D_OUT = 64

NC = 2
NS = 16
TPN = N_PAD // NS
ZR = 64

MP_K = 125
MP_NB = (E // NS) // MP_K

DEG_W = 128

_mesh = plsc.VectorSubcoreMesh(core_axis_name="c", subcore_axis_name="s")


def _sc_degree(dst2d):
    wnb = MP_NB // NC

    @functools.partial(
        pl.kernel,
        out_type=jax.ShapeDtypeStruct((NC * N_PAD, DEG_W), jnp.float32),
        mesh=_mesh,
        scratch_types=[
            pltpu.VMEM((wnb, MP_K), jnp.int32),
            pltpu.VMEM((MP_K, DEG_W), jnp.float32),
            pltpu.VMEM((ZR, DEG_W), jnp.float32),
            pltpu.VMEM_SHARED((N_PAD, DEG_W), jnp.float32),
        ],
    )
    def k(dst_hbm, out_hbm, dstv, ones, zbuf, hist_sh):
        c = lax.axis_index("c")
        s = lax.axis_index("s")
        w = c * NS + s

        def fill_ones(r, _):
            def col(j, _):
                ones[r, pl.ds(j * 16, 16)] = jnp.ones((16,), jnp.float32)
                return 0
            return lax.fori_loop(0, DEG_W // 16, col, 0)

        lax.fori_loop(0, MP_K, fill_ones, 0)

        def fill_zero(r, _):
            def col(j, _):
                zbuf[r, pl.ds(j * 16, 16)] = jnp.zeros((16,), jnp.float32)
                return 0
            return lax.fori_loop(0, DEG_W // 16, col, 0)

        lax.fori_loop(0, ZR, fill_zero, 0)

        def zero_blk(j, _):
            pltpu.sync_copy(zbuf, hist_sh.at[pl.ds(s * TPN + j * ZR, ZR)])
            return 0

        lax.fori_loop(0, TPN // ZR, zero_blk, 0)
        pltpu.sync_copy(dst_hbm.at[pl.ds(w * wnb, wnb)], dstv)
        plsc.subcore_barrier()

        def scat(b, _):
            pltpu.sync_copy(ones, hist_sh.at[dstv.at[b]], add=True)
            return 0

        lax.fori_loop(0, wnb, scat, 0)
        plsc.subcore_barrier()

        def wout(j, _):
            pltpu.sync_copy(hist_sh.at[pl.ds(s * TPN + j * ZR, ZR)], zbuf)
            base = pl.multiple_of(c * N_PAD + s * TPN + j * ZR, ZR)
            pltpu.sync_copy(zbuf, out_hbm.at[pl.ds(base, ZR)])
            return 0

        lax.fori_loop(0, TPN // ZR, wout, 0)

    return k(dst2d)


ICH = 8


def _ring_row(b):
    return ((b // ICH) % 2) * ICH + (b % ICH)


def _mp_pipeline(hp_hbm, acc_sh, srcr, dstr, rows0, rows1,
                 sem0, sem1, semi_s, semi_d, src_hbm, dst_hbm,
                 base_row, wrows):
    pltpu.sync_copy(src_hbm.at[pl.ds(base_row, ICH)], srcr.at[pl.ds(0, ICH)])
    pltpu.sync_copy(dst_hbm.at[pl.ds(base_row, ICH)], dstr.at[pl.ds(0, ICH)])
    pltpu.async_copy(hp_hbm.at[srcr.at[0]], rows0, sem0)

    def pair(p, _):
        b0 = 2 * p
        b1 = b0 + 1

        @pl.when(jnp.logical_and(b0 % ICH == 0, b0 + ICH < wrows))
        def _():
            cn = b0 // ICH + 1
            off = pl.multiple_of((cn % 2) * ICH, ICH)
            pltpu.async_copy(src_hbm.at[pl.ds(base_row + cn * ICH, ICH)],
                             srcr.at[pl.ds(off, ICH)], semi_s)
            pltpu.async_copy(dst_hbm.at[pl.ds(base_row + cn * ICH, ICH)],
                             dstr.at[pl.ds(off, ICH)], semi_d)

        rr0 = _ring_row(b0)
        rr1 = _ring_row(b1)
        pltpu.async_copy(hp_hbm.at[srcr.at[rr1]], rows1, sem1)
        pltpu.make_async_copy(hp_hbm.at[srcr.at[rr0]], rows0, sem0).wait()
        pltpu.sync_copy(rows0, acc_sh.at[dstr.at[rr0]], add=True)

        @pl.when(b0 + 2 < wrows)
        def _():
            @pl.when((b0 + 2) % ICH == 0)
            def _():
                pltpu.make_async_copy(
                    src_hbm.at[pl.ds(base_row, ICH)],
                    srcr.at[pl.ds(0, ICH)], semi_s).wait()
                pltpu.make_async_copy(
                    dst_hbm.at[pl.ds(base_row, ICH)],
                    dstr.at[pl.ds(0, ICH)], semi_d).wait()

            pltpu.async_copy(hp_hbm.at[srcr.at[_ring_row(b0 + 2)]],
                             rows0, sem0)

        pltpu.make_async_copy(hp_hbm.at[srcr.at[rr1]], rows1, sem1).wait()
        pltpu.sync_copy(rows1, acc_sh.at[dstr.at[rr1]], add=True)
        return 0

    lax.fori_loop(0, wrows // 2, pair, 0)


def _zero_fill(zbuf, dh):
    def fill_zero(r, _):
        def col(j, _):
            zbuf[r, pl.ds(j * 16, 16)] = jnp.zeros((16,), jnp.float32)
            return 0
        return lax.fori_loop(0, dh // 16, col, 0)

    lax.fori_loop(0, ZR, fill_zero, 0)


def _sc_msgpass(hp_lo, hp_hi, src2d, dst2d, dh):
    @functools.partial(
        pl.kernel,
        out_type=[jax.ShapeDtypeStruct((N_PAD, dh), jnp.float32)] * 2,
        mesh=_mesh,
        scratch_types=[
            pltpu.VMEM((2 * ICH, MP_K), jnp.int32),
            pltpu.VMEM((2 * ICH, MP_K), jnp.int32),
            pltpu.VMEM((MP_K, dh), jnp.float32),
            pltpu.VMEM((MP_K, dh), jnp.float32),
            pltpu.VMEM((ZR, dh), jnp.float32),
            pltpu.VMEM_SHARED((N_PAD, dh), jnp.float32),
            pltpu.SemaphoreType.DMA,
            pltpu.SemaphoreType.DMA,
            pltpu.SemaphoreType.DMA,
            pltpu.SemaphoreType.DMA,
        ],
    )
    def k(lo_hbm, hi_hbm, src_hbm, dst_hbm, olo_hbm, ohi_hbm,
          srcr, dstr, rows0, rows1, zbuf, acc_sh, sem0, sem1, semi_s, semi_d):
        c = lax.axis_index("c")
        s = lax.axis_index("s")

        _zero_fill(zbuf, dh)

        def zero_blk(j, _):
            pltpu.sync_copy(zbuf, acc_sh.at[pl.ds(s * TPN + j * ZR, ZR)])
            return 0

        lax.fori_loop(0, TPN // ZR, zero_blk, 0)
        plsc.subcore_barrier()

        def body(hp_hbm, out_hbm):
            _mp_pipeline(hp_hbm, acc_sh, srcr, dstr, rows0, rows1,
                         sem0, sem1, semi_s, semi_d, src_hbm, dst_hbm,
                         s * MP_NB, MP_NB)
            plsc.subcore_barrier()

            def wout(j, _):
                pltpu.sync_copy(acc_sh.at[pl.ds(s * TPN + j * ZR, ZR)], zbuf)
                pltpu.sync_copy(zbuf, out_hbm.at[pl.ds(s * TPN + j * ZR, ZR)])
                return 0

            lax.fori_loop(0, TPN // ZR, wout, 0)

        @pl.when(c == 0)
        def _():
            body(lo_hbm, olo_hbm)

        @pl.when(c == 1)
        def _():
            body(hi_hbm, ohi_hbm)

    return k(hp_lo, hp_hi, src2d, dst2d)


def _sc_msgpass_edges(hp, src2d, dst2d):
    dh = 128
    wnb = MP_NB // NC

    @functools.partial(
        pl.kernel,
        out_type=jax.ShapeDtypeStruct((NC * N_PAD, dh), jnp.float32),
        mesh=_mesh,
        scratch_types=[
            pltpu.VMEM((2 * ICH, MP_K), jnp.int32),
            pltpu.VMEM((2 * ICH, MP_K), jnp.int32),
            pltpu.VMEM((MP_K, dh), jnp.float32),
            pltpu.VMEM((MP_K, dh), jnp.float32),
            pltpu.VMEM((ZR, dh), jnp.float32),
            pltpu.VMEM_SHARED((N_PAD, dh), jnp.float32),
            pltpu.SemaphoreType.DMA,
            pltpu.SemaphoreType.DMA,
            pltpu.SemaphoreType.DMA,
            pltpu.SemaphoreType.DMA,
        ],
    )
    def k(hp_hbm, src_hbm, dst_hbm, out_hbm,
          srcr, dstr, rows0, rows1, zbuf, acc_sh,
          sem0, sem1, semi_s, semi_d):
        c = lax.axis_index("c")
        s = lax.axis_index("s")
        w = c * NS + s

        _zero_fill(zbuf, dh)

        def zero_blk(j, _):
            pltpu.sync_copy(zbuf, acc_sh.at[pl.ds(s * TPN + j * ZR, ZR)])
            return 0

        lax.fori_loop(0, TPN // ZR, zero_blk, 0)
        plsc.subcore_barrier()

        _mp_pipeline(hp_hbm, acc_sh, srcr, dstr, rows0, rows1,
                     sem0, sem1, semi_s, semi_d, src_hbm, dst_hbm,
                     w * wnb, wnb)
        plsc.subcore_barrier()

        def wout(j, _):
            pltpu.sync_copy(acc_sh.at[pl.ds(s * TPN + j * ZR, ZR)], zbuf)
            base = pl.multiple_of(c * N_PAD + s * TPN + j * ZR, ZR)
            pltpu.sync_copy(zbuf, out_hbm.at[pl.ds(base, ZR)])
            return 0

        lax.fori_loop(0, TPN // ZR, wout, 0)

    return k(hp, src2d, dst2d)


_R = 1024
_G = N_PAD // _R


def _dinv_block(dega_ref, degb_ref):
    deg = dega_ref[:, 0:1] + degb_ref[:, 0:1] + 1.0
    return lax.rsqrt(deg)


def _tc0_body(x_ref, w_ref, h_ref):
    h_ref[...] = jnp.dot(x_ref[...], w_ref[...],
                         preferred_element_type=jnp.float32)


def _tc1_body(h_ref, dega_ref, degb_ref, lo_ref, hi_ref):
    dinv = _dinv_block(dega_ref, degb_ref)
    hp = h_ref[...] * dinv
    lo_ref[...] = hp[:, : D_HID // 2]
    hi_ref[...] = hp[:, D_HID // 2:]


def _tc2_body(alo_ref, ahi_ref, plo_ref, phi_ref, dega_ref, degb_ref,
              b1_ref, w2_ref, o_ref):
    dinv = _dinv_block(dega_ref, degb_ref)
    zlo = jnp.maximum(dinv * (alo_ref[...] + plo_ref[...])
                      + b1_ref[0:1, : D_HID // 2], 0.0)
    zhi = jnp.maximum(dinv * (ahi_ref[...] + phi_ref[...])
                      + b1_ref[0:1, D_HID // 2:], 0.0)
    z = jnp.concatenate([zlo, zhi], axis=1)
    h2 = jnp.dot(z, w2_ref[...], preferred_element_type=jnp.float32)
    h2p = h2 * dinv
    o_ref[...] = jnp.concatenate(
        [h2p, jnp.zeros((h2p.shape[0], 128 - D_OUT), jnp.float32)], axis=1)


def _tc3_body(a0_ref, a1_ref, p_ref, dega_ref, degb_ref, b2_ref, out_ref):
    dinv = _dinv_block(dega_ref, degb_ref)
    acc = a0_ref[:, :D_OUT] + a1_ref[:, :D_OUT] + p_ref[:, :D_OUT]
    out_ref[...] = dinv * acc + b2_ref[0:1, :]


def _rows_spec(cols):
    return pl.BlockSpec((_R, cols), lambda i: (i, 0))


_DEG_SPECS = [pl.BlockSpec((_R, DEG_W), lambda i: (i, 0)),
              pl.BlockSpec((_R, DEG_W), lambda i: (i + _G, 0))]


def _full_spec(r, c):
    return pl.BlockSpec((r, c), lambda i: (0, 0))


def kernel(x, edge_index, W1, b1, W2, b2):
    src = edge_index[0].astype(jnp.int32)
    dst = edge_index[1].astype(jnp.int32)
    src_mp = src.reshape(E // MP_K, MP_K)
    dst_mp = dst.reshape(E // MP_K, MP_K)
    xp = jnp.pad(x, ((0, N_PAD - N), (0, 0)))
    b1r = b1.reshape(1, D_HID)
    b2r = b2.reshape(1, D_OUT)

    deg_hist = _sc_degree(dst_mp)

    h1 = pl.pallas_call(
        _tc0_body,
        grid=(_G,),
        in_specs=[_rows_spec(D_IN), _full_spec(D_IN, D_HID)],
        out_specs=_rows_spec(D_HID),
        out_shape=jax.ShapeDtypeStruct((N_PAD, D_HID), jnp.float32),
    )(xp, W1)

    h1p_lo, h1p_hi = pl.pallas_call(
        _tc1_body,
        grid=(_G,),
        in_specs=[_rows_spec(D_HID)] + _DEG_SPECS,
        out_specs=[_rows_spec(D_HID // 2)] * 2,
        out_shape=[jax.ShapeDtypeStruct((N_PAD, D_HID // 2), jnp.float32)] * 2,
    )(h1, deg_hist, deg_hist)

    acc1_lo, acc1_hi = _sc_msgpass(h1p_lo, h1p_hi, src_mp, dst_mp, D_HID // 2)

    h2p = pl.pallas_call(
        _tc2_body,
        grid=(_G,),
        in_specs=[
            _rows_spec(D_HID // 2), _rows_spec(D_HID // 2),
            _rows_spec(D_HID // 2), _rows_spec(D_HID // 2),
        ] + _DEG_SPECS + [
            _full_spec(1, D_HID),
            _full_spec(D_HID, D_OUT),
        ],
        out_specs=_rows_spec(128),
        out_shape=jax.ShapeDtypeStruct((N_PAD, 128), jnp.float32),
    )(acc1_lo, acc1_hi, h1p_lo, h1p_hi, deg_hist, deg_hist, b1r, W2)

    acc2 = _sc_msgpass_edges(h2p, src_mp, dst_mp)

    out = pl.pallas_call(
        _tc3_body,
        grid=(_G,),
        in_specs=[
            pl.BlockSpec((_R, 128), lambda i: (i, 0)),
            pl.BlockSpec((_R, 128), lambda i: (i + _G, 0)),
            _rows_spec(128),
        ] + _DEG_SPECS + [
            _full_spec(1, D_OUT),
        ],
        out_specs=_rows_spec(D_OUT),
        out_shape=jax.ShapeDtypeStruct((N_PAD, D_OUT), jnp.float32),
    )(acc2, acc2, h2p, deg_hist, deg_hist, b2r)

    return out[:N]


# speedup vs baseline: 19.2051x; 1.0018x over previous
import functools

import jax
import jax.numpy as jnp
from jax import lax
from jax.experimental import pallas as pl
from jax.experimental.pallas import tpu as pltpu
from jax.experimental.pallas import tpu_sc as plsc

N = 10000
N_PAD = 10240
E = 160000
D_IN = 256
D_HID = 256
D_OUT = 64

NC = 2
NS = 16
TPN = N_PAD // NS
ZR = 64

MP_K = 125
MP_NB = (E // NS) // MP_K

DEG_W = 128

_mesh = plsc.VectorSubcoreMesh(core_axis_name="c", subcore_axis_name="s")


def _sc_degree(dst2d):
    wnb = MP_NB // NC

    @functools.partial(
        pl.kernel,
        out_type=jax.ShapeDtypeStruct((NC * N_PAD, DEG_W), jnp.float32),
        mesh=_mesh,
        scratch_types=[
            pltpu.VMEM((wnb, MP_K), jnp.int32),
            pltpu.VMEM((MP_K, DEG_W), jnp.float32),
            pltpu.VMEM((ZR, DEG_W), jnp.float32),
            pltpu.VMEM_SHARED((N_PAD, DEG_W), jnp.float32),
        ],
    )
    def k(dst_hbm, out_hbm, dstv, ones, zbuf, hist_sh):
        c = lax.axis_index("c")
        s = lax.axis_index("s")
        w = c * NS + s

        def fill_ones(r, _):
            def col(j, _):
                ones[r, pl.ds(j * 16, 16)] = jnp.ones((16,), jnp.float32)
                return 0
            return lax.fori_loop(0, DEG_W // 16, col, 0)

        lax.fori_loop(0, MP_K, fill_ones, 0)

        def fill_zero(r, _):
            def col(j, _):
                zbuf[r, pl.ds(j * 16, 16)] = jnp.zeros((16,), jnp.float32)
                return 0
            return lax.fori_loop(0, DEG_W // 16, col, 0)

        lax.fori_loop(0, ZR, fill_zero, 0)

        def zero_blk(j, _):
            pltpu.sync_copy(zbuf, hist_sh.at[pl.ds(s * TPN + j * ZR, ZR)])
            return 0

        lax.fori_loop(0, TPN // ZR, zero_blk, 0)
        pltpu.sync_copy(dst_hbm.at[pl.ds(w * wnb, wnb)], dstv)
        plsc.subcore_barrier()

        def scat(b, _):
            pltpu.sync_copy(ones, hist_sh.at[dstv.at[b]], add=True)
            return 0

        lax.fori_loop(0, wnb, scat, 0)
        plsc.subcore_barrier()

        def wout(j, _):
            pltpu.sync_copy(hist_sh.at[pl.ds(s * TPN + j * ZR, ZR)], zbuf)
            base = pl.multiple_of(c * N_PAD + s * TPN + j * ZR, ZR)
            pltpu.sync_copy(zbuf, out_hbm.at[pl.ds(base, ZR)])
            return 0

        lax.fori_loop(0, TPN // ZR, wout, 0)

    return k(dst2d)


ICH = 8


def _ring_row(b):
    return ((b // ICH) % 2) * ICH + (b % ICH)


def _mp_pipeline(hp_hbm, acc_sh, srcr, dstr, rows0, rows1,
                 sem0, sem1, semi_s, semi_d, src_hbm, dst_hbm,
                 base_row, wrows):
    pltpu.sync_copy(src_hbm.at[pl.ds(base_row, ICH)], srcr.at[pl.ds(0, ICH)])
    pltpu.sync_copy(dst_hbm.at[pl.ds(base_row, ICH)], dstr.at[pl.ds(0, ICH)])
    pltpu.async_copy(hp_hbm.at[srcr.at[0]], rows0, sem0)

    def pair(p, _):
        b0 = 2 * p
        b1 = b0 + 1

        @pl.when(jnp.logical_and(b0 % ICH == 0, b0 + ICH < wrows))
        def _():
            cn = b0 // ICH + 1
            off = pl.multiple_of((cn % 2) * ICH, ICH)
            pltpu.async_copy(src_hbm.at[pl.ds(base_row + cn * ICH, ICH)],
                             srcr.at[pl.ds(off, ICH)], semi_s)
            pltpu.async_copy(dst_hbm.at[pl.ds(base_row + cn * ICH, ICH)],
                             dstr.at[pl.ds(off, ICH)], semi_d)

        rr0 = _ring_row(b0)
        rr1 = _ring_row(b1)
        pltpu.async_copy(hp_hbm.at[srcr.at[rr1]], rows1, sem1)
        pltpu.make_async_copy(hp_hbm.at[srcr.at[rr0]], rows0, sem0).wait()
        pltpu.sync_copy(rows0, acc_sh.at[dstr.at[rr0]], add=True)

        @pl.when(b0 + 2 < wrows)
        def _():
            @pl.when((b0 + 2) % ICH == 0)
            def _():
                pltpu.make_async_copy(
                    src_hbm.at[pl.ds(base_row, ICH)],
                    srcr.at[pl.ds(0, ICH)], semi_s).wait()
                pltpu.make_async_copy(
                    dst_hbm.at[pl.ds(base_row, ICH)],
                    dstr.at[pl.ds(0, ICH)], semi_d).wait()

            pltpu.async_copy(hp_hbm.at[srcr.at[_ring_row(b0 + 2)]],
                             rows0, sem0)

        pltpu.make_async_copy(hp_hbm.at[srcr.at[rr1]], rows1, sem1).wait()
        pltpu.sync_copy(rows1, acc_sh.at[dstr.at[rr1]], add=True)
        return 0

    lax.fori_loop(0, wrows // 2, pair, 0)


def _zero_fill(zbuf, dh, dt=jnp.float32):
    if dt == jnp.bfloat16:
        def fill_zero(r, _):
            def col(j, _):
                zbuf[pl.ds(2 * r, 2), pl.ds(j * 16, 16)] = (
                    jnp.zeros((2, 16), dt))
                return 0
            return lax.fori_loop(0, dh // 16, col, 0)

        lax.fori_loop(0, ZR // 2, fill_zero, 0)
    else:
        def fill_zero(r, _):
            def col(j, _):
                zbuf[r, pl.ds(j * 16, 16)] = jnp.zeros((16,), dt)
                return 0
            return lax.fori_loop(0, dh // 16, col, 0)

        lax.fori_loop(0, ZR, fill_zero, 0)


def _zero_fill3(zbuf):
    def fill_zero(r, _):
        def col(j, _):
            zbuf[r, pl.ds(0, 2), pl.ds(j * 16, 16)] = (
                jnp.zeros((2, 16), jnp.bfloat16))
            return 0
        return lax.fori_loop(0, 8, col, 0)

    lax.fori_loop(0, ZR, fill_zero, 0)


def _sc_msgpass(hp_lo, hp_hi, src2d, dst2d, dh, dt=jnp.float32):
    @functools.partial(
        pl.kernel,
        out_type=[jax.ShapeDtypeStruct((N_PAD, dh), dt)] * 2,
        mesh=_mesh,
        scratch_types=[
            pltpu.VMEM((2 * ICH, MP_K), jnp.int32),
            pltpu.VMEM((2 * ICH, MP_K), jnp.int32),
            pltpu.VMEM((MP_K, dh), dt),
            pltpu.VMEM((MP_K, dh), dt),
            pltpu.VMEM((ZR, dh), dt),
            pltpu.VMEM_SHARED((N_PAD, dh), dt),
            pltpu.SemaphoreType.DMA,
            pltpu.SemaphoreType.DMA,
            pltpu.SemaphoreType.DMA,
            pltpu.SemaphoreType.DMA,
        ],
    )
    def k(lo_hbm, hi_hbm, src_hbm, dst_hbm, olo_hbm, ohi_hbm,
          srcr, dstr, rows0, rows1, zbuf, acc_sh, sem0, sem1, semi_s, semi_d):
        c = lax.axis_index("c")
        s = lax.axis_index("s")

        _zero_fill(zbuf, dh, dt)

        def zero_blk(j, _):
            pltpu.sync_copy(zbuf, acc_sh.at[pl.ds(s * TPN + j * ZR, ZR)])
            return 0

        lax.fori_loop(0, TPN // ZR, zero_blk, 0)
        plsc.subcore_barrier()

        def body(hp_hbm, out_hbm):
            _mp_pipeline(hp_hbm, acc_sh, srcr, dstr, rows0, rows1,
                         sem0, sem1, semi_s, semi_d, src_hbm, dst_hbm,
                         s * MP_NB, MP_NB)
            plsc.subcore_barrier()

            def wout(j, _):
                pltpu.sync_copy(acc_sh.at[pl.ds(s * TPN + j * ZR, ZR)], zbuf)
                pltpu.sync_copy(zbuf, out_hbm.at[pl.ds(s * TPN + j * ZR, ZR)])
                return 0

            lax.fori_loop(0, TPN // ZR, wout, 0)

        @pl.when(c == 0)
        def _():
            body(lo_hbm, olo_hbm)

        @pl.when(c == 1)
        def _():
            body(hi_hbm, ohi_hbm)

    return k(hp_lo, hp_hi, src2d, dst2d)


def _sc_msgpass_edges(hp, src2d, dst2d, dt=jnp.float32):
    row_shape = tuple(hp.shape[1:])
    wnb = MP_NB // NC

    @functools.partial(
        pl.kernel,
        out_type=jax.ShapeDtypeStruct((NC * N_PAD,) + row_shape, dt),
        mesh=_mesh,
        scratch_types=[
            pltpu.VMEM((2 * ICH, MP_K), jnp.int32),
            pltpu.VMEM((2 * ICH, MP_K), jnp.int32),
            pltpu.VMEM((MP_K,) + row_shape, dt),
            pltpu.VMEM((MP_K,) + row_shape, dt),
            pltpu.VMEM((ZR,) + row_shape, dt),
            pltpu.VMEM_SHARED((N_PAD,) + row_shape, dt),
            pltpu.SemaphoreType.DMA,
            pltpu.SemaphoreType.DMA,
            pltpu.SemaphoreType.DMA,
            pltpu.SemaphoreType.DMA,
        ],
    )
    def k(hp_hbm, src_hbm, dst_hbm, out_hbm,
          srcr, dstr, rows0, rows1, zbuf, acc_sh,
          sem0, sem1, semi_s, semi_d):
        c = lax.axis_index("c")
        s = lax.axis_index("s")
        w = c * NS + s

        if dt == jnp.bfloat16:
            _zero_fill3(zbuf)
        else:
            _zero_fill(zbuf, row_shape[0], dt)

        def zero_blk(j, _):
            pltpu.sync_copy(zbuf, acc_sh.at[pl.ds(s * TPN + j * ZR, ZR)])
            return 0

        lax.fori_loop(0, TPN // ZR, zero_blk, 0)
        plsc.subcore_barrier()

        _mp_pipeline(hp_hbm, acc_sh, srcr, dstr, rows0, rows1,
                     sem0, sem1, semi_s, semi_d, src_hbm, dst_hbm,
                     w * wnb, wnb)
        plsc.subcore_barrier()

        def wout(j, _):
            pltpu.sync_copy(acc_sh.at[pl.ds(s * TPN + j * ZR, ZR)], zbuf)
            base = pl.multiple_of(c * N_PAD + s * TPN + j * ZR, ZR)
            pltpu.sync_copy(zbuf, out_hbm.at[pl.ds(base, ZR)])
            return 0

        lax.fori_loop(0, TPN // ZR, wout, 0)

    return k(hp, src2d, dst2d)


_R = 1024
_G = N_PAD // _R


def _dinv_block(dega_ref, degb_ref):
    deg = dega_ref[:, 0:1] + degb_ref[:, 0:1] + 1.0
    return lax.rsqrt(deg)


def _tc0_body(x_ref, w_ref, h_ref):
    h_ref[...] = jnp.dot(x_ref[...], w_ref[...],
                         preferred_element_type=jnp.float32)


def _tc1_body(h_ref, dega_ref, degb_ref, lo_ref, hi_ref):
    dinv = _dinv_block(dega_ref, degb_ref)
    hp = h_ref[...] * dinv
    lo_ref[...] = hp[:, : D_HID // 2]
    hi_ref[...] = hp[:, D_HID // 2:]


def _tc2_body(alo_ref, ahi_ref, plo_ref, phi_ref, dega_ref, degb_ref,
              b1_ref, w2_ref, o_ref):
    dinv = _dinv_block(dega_ref, degb_ref)
    zlo = jnp.maximum(dinv * (alo_ref[...] + plo_ref[...])
                      + b1_ref[0:1, : D_HID // 2], 0.0)
    zhi = jnp.maximum(dinv * (ahi_ref[...] + phi_ref[...])
                      + b1_ref[0:1, D_HID // 2:], 0.0)
    z = jnp.concatenate([zlo, zhi], axis=1)
    h2 = jnp.dot(z, w2_ref[...], preferred_element_type=jnp.float32)
    h2p = h2 * dinv
    o_ref[...] = jnp.concatenate(
        [h2p, jnp.zeros((h2p.shape[0], 128 - D_OUT), jnp.float32)], axis=1)


def _tc3_body(a0_ref, a1_ref, p_ref, dega_ref, degb_ref, b2_ref, out_ref):
    dinv = _dinv_block(dega_ref, degb_ref)
    acc = (a0_ref[:, :D_OUT].astype(jnp.float32)
           + a1_ref[:, :D_OUT].astype(jnp.float32)
           + p_ref[:, :D_OUT].astype(jnp.float32))
    out_ref[...] = dinv * acc + b2_ref[0:1, :]


def _rows_spec(cols):
    return pl.BlockSpec((_R, cols), lambda i: (i, 0))


_DEG_SPECS = [pl.BlockSpec((_R, DEG_W), lambda i: (i, 0)),
              pl.BlockSpec((_R, DEG_W), lambda i: (i + _G, 0))]


def _full_spec(r, c):
    return pl.BlockSpec((r, c), lambda i: (0, 0))


def kernel(x, edge_index, W1, b1, W2, b2):
    src = edge_index[0].astype(jnp.int32)
    dst = edge_index[1].astype(jnp.int32)
    src_mp = src.reshape(E // MP_K, MP_K)
    dst_mp = dst.reshape(E // MP_K, MP_K)
    xp = jnp.pad(x, ((0, N_PAD - N), (0, 0)))
    b1r = b1.reshape(1, D_HID)
    b2r = b2.reshape(1, D_OUT)

    deg_hist = _sc_degree(dst_mp)

    h1 = pl.pallas_call(
        _tc0_body,
        grid=(_G,),
        in_specs=[_rows_spec(D_IN), _full_spec(D_IN, D_HID)],
        out_specs=_rows_spec(D_HID),
        out_shape=jax.ShapeDtypeStruct((N_PAD, D_HID), jnp.float32),
    )(xp, W1)

    h1p_lo, h1p_hi = pl.pallas_call(
        _tc1_body,
        grid=(_G,),
        in_specs=[_rows_spec(D_HID)] + _DEG_SPECS,
        out_specs=[_rows_spec(D_HID // 2)] * 2,
        out_shape=[jax.ShapeDtypeStruct((N_PAD, D_HID // 2), jnp.float32)] * 2,
    )(h1, deg_hist, deg_hist)

    acc1_lo, acc1_hi = _sc_msgpass(h1p_lo, h1p_hi, src_mp, dst_mp, D_HID // 2)

    h2p = pl.pallas_call(
        _tc2_body,
        grid=(_G,),
        in_specs=[
            _rows_spec(D_HID // 2), _rows_spec(D_HID // 2),
            _rows_spec(D_HID // 2), _rows_spec(D_HID // 2),
        ] + _DEG_SPECS + [
            _full_spec(1, D_HID),
            _full_spec(D_HID, D_OUT),
        ],
        out_specs=_rows_spec(128),
        out_shape=jax.ShapeDtypeStruct((N_PAD, 128), jnp.float32),
    )(acc1_lo, acc1_hi, h1p_lo, h1p_hi, deg_hist, deg_hist, b1r, W2)

    acc2 = _sc_msgpass_edges(h2p, src_mp, dst_mp)

    out = pl.pallas_call(
        _tc3_body,
        grid=(_G,),
        in_specs=[
            pl.BlockSpec((_R, 128), lambda i: (i, 0)),
            pl.BlockSpec((_R, 128), lambda i: (i + _G, 0)),
            _rows_spec(128),
        ] + _DEG_SPECS + [
            _full_spec(1, D_OUT),
        ],
        out_specs=_rows_spec(D_OUT),
        out_shape=jax.ShapeDtypeStruct((N_PAD, D_OUT), jnp.float32),
    )(acc2, acc2, h2p, deg_hist, deg_hist, b2r)

    return out[:N]


# speedup vs baseline: 20.6513x; 1.0753x over previous
import functools

import jax
import jax.numpy as jnp
from jax import lax
from jax.experimental import pallas as pl
from jax.experimental.pallas import tpu as pltpu
from jax.experimental.pallas import tpu_sc as plsc

N = 10000
N_PAD = 10240
E = 160000
D_IN = 256
D_HID = 256
D_OUT = 64

NC = 2
NS = 16
TPN = N_PAD // NS
ZR = 64

MP_K = 125
MP_NB = (E // NS) // MP_K

DEG_W = 8

_mesh = plsc.VectorSubcoreMesh(core_axis_name="c", subcore_axis_name="s")
_NOTILE = pltpu.CompilerParams(use_tc_tiling_on_sc=False)


def _sc_degree(dst2d):
    wnb = MP_NB // NC

    @functools.partial(
        pl.kernel,
        out_type=jax.ShapeDtypeStruct((NC * N_PAD, DEG_W), jnp.float32),
        mesh=_mesh,
        compiler_params=_NOTILE,
        scratch_types=[
            pltpu.VMEM((wnb, MP_K), jnp.int32),
            pltpu.VMEM((128, DEG_W), jnp.float32),
            pltpu.VMEM((ZR, DEG_W), jnp.float32),
            pltpu.VMEM_SHARED((N_PAD, DEG_W), jnp.float32),
        ],
    )
    def k(dst_hbm, out_hbm, dstv, ones, zbuf, hist_sh):
        c = lax.axis_index("c")
        s = lax.axis_index("s")
        w = c * NS + s

        def fill_ones(r, _):
            ones[pl.ds(r * 16, 16), pl.ds(0, DEG_W)] = (
                jnp.ones((16, DEG_W), jnp.float32))
            return 0

        lax.fori_loop(0, 128 // 16, fill_ones, 0)

        def fill_zero(r, _):
            zbuf[pl.ds(r * 16, 16), pl.ds(0, DEG_W)] = (
                jnp.zeros((16, DEG_W), jnp.float32))
            return 0

        lax.fori_loop(0, ZR // 16, fill_zero, 0)

        def zero_blk(j, _):
            pltpu.sync_copy(zbuf, hist_sh.at[pl.ds(s * TPN + j * ZR, ZR)])
            return 0

        lax.fori_loop(0, TPN // ZR, zero_blk, 0)
        pltpu.sync_copy(dst_hbm.at[pl.ds(w * wnb, wnb)], dstv)
        plsc.subcore_barrier()

        def scat(b, _):
            pltpu.sync_copy(ones.at[pl.ds(0, MP_K)],
                            hist_sh.at[dstv.at[b]], add=True)
            return 0

        lax.fori_loop(0, wnb, scat, 0)
        plsc.subcore_barrier()

        def wout(j, _):
            pltpu.sync_copy(hist_sh.at[pl.ds(s * TPN + j * ZR, ZR)], zbuf)
            base = pl.multiple_of(c * N_PAD + s * TPN + j * ZR, ZR)
            pltpu.sync_copy(zbuf, out_hbm.at[pl.ds(base, ZR)])
            return 0

        lax.fori_loop(0, TPN // ZR, wout, 0)

    return k(dst2d)


ICH = 8


def _ring_row(b):
    return ((b // ICH) % 2) * ICH + (b % ICH)


def _mp_pipeline(hp_hbm, acc_sh, srcr, dstr, rows0, rows1,
                 sem0, sem1, semi_s, semi_d, src_hbm, dst_hbm,
                 base_row, wrows):
    pltpu.sync_copy(src_hbm.at[pl.ds(base_row, ICH)], srcr.at[pl.ds(0, ICH)])
    pltpu.sync_copy(dst_hbm.at[pl.ds(base_row, ICH)], dstr.at[pl.ds(0, ICH)])
    pltpu.async_copy(hp_hbm.at[srcr.at[0]], rows0, sem0)

    def pair(p, _):
        b0 = 2 * p
        b1 = b0 + 1

        @pl.when(jnp.logical_and(b0 % ICH == 0, b0 + ICH < wrows))
        def _():
            cn = b0 // ICH + 1
            off = pl.multiple_of((cn % 2) * ICH, ICH)
            pltpu.async_copy(src_hbm.at[pl.ds(base_row + cn * ICH, ICH)],
                             srcr.at[pl.ds(off, ICH)], semi_s)
            pltpu.async_copy(dst_hbm.at[pl.ds(base_row + cn * ICH, ICH)],
                             dstr.at[pl.ds(off, ICH)], semi_d)

        rr0 = _ring_row(b0)
        rr1 = _ring_row(b1)
        pltpu.async_copy(hp_hbm.at[srcr.at[rr1]], rows1, sem1)
        pltpu.make_async_copy(hp_hbm.at[srcr.at[rr0]], rows0, sem0).wait()
        pltpu.sync_copy(rows0, acc_sh.at[dstr.at[rr0]], add=True)

        @pl.when(b0 + 2 < wrows)
        def _():
            @pl.when((b0 + 2) % ICH == 0)
            def _():
                pltpu.make_async_copy(
                    src_hbm.at[pl.ds(base_row, ICH)],
                    srcr.at[pl.ds(0, ICH)], semi_s).wait()
                pltpu.make_async_copy(
                    dst_hbm.at[pl.ds(base_row, ICH)],
                    dstr.at[pl.ds(0, ICH)], semi_d).wait()

            pltpu.async_copy(hp_hbm.at[srcr.at[_ring_row(b0 + 2)]],
                             rows0, sem0)

        pltpu.make_async_copy(hp_hbm.at[srcr.at[rr1]], rows1, sem1).wait()
        pltpu.sync_copy(rows1, acc_sh.at[dstr.at[rr1]], add=True)
        return 0

    lax.fori_loop(0, wrows // 2, pair, 0)


def _zero_fill(zbuf, dh, dt=jnp.float32):
    if dt == jnp.bfloat16:
        def fill_zero(r, _):
            def col(j, _):
                zbuf[pl.ds(2 * r, 2), pl.ds(j * 16, 16)] = (
                    jnp.zeros((2, 16), dt))
                return 0
            return lax.fori_loop(0, dh // 16, col, 0)

        lax.fori_loop(0, ZR // 2, fill_zero, 0)
    else:
        def fill_zero(r, _):
            def col(j, _):
                zbuf[r, pl.ds(j * 16, 16)] = jnp.zeros((16,), dt)
                return 0
            return lax.fori_loop(0, dh // 16, col, 0)

        lax.fori_loop(0, ZR, fill_zero, 0)


def _zero_fill3(zbuf):
    def fill_zero(r, _):
        def col(j, _):
            zbuf[r, pl.ds(0, 2), pl.ds(j * 16, 16)] = (
                jnp.zeros((2, 16), jnp.bfloat16))
            return 0
        return lax.fori_loop(0, 8, col, 0)

    lax.fori_loop(0, ZR, fill_zero, 0)


def _sc_msgpass(hp_lo, hp_hi, src2d, dst2d, dh, dt=jnp.float32):
    @functools.partial(
        pl.kernel,
        out_type=[jax.ShapeDtypeStruct((N_PAD, dh), dt)] * 2,
        mesh=_mesh,
        scratch_types=[
            pltpu.VMEM((2 * ICH, MP_K), jnp.int32),
            pltpu.VMEM((2 * ICH, MP_K), jnp.int32),
            pltpu.VMEM((MP_K, dh), dt),
            pltpu.VMEM((MP_K, dh), dt),
            pltpu.VMEM((ZR, dh), dt),
            pltpu.VMEM_SHARED((N_PAD, dh), dt),
            pltpu.SemaphoreType.DMA,
            pltpu.SemaphoreType.DMA,
            pltpu.SemaphoreType.DMA,
            pltpu.SemaphoreType.DMA,
        ],
    )
    def k(lo_hbm, hi_hbm, src_hbm, dst_hbm, olo_hbm, ohi_hbm,
          srcr, dstr, rows0, rows1, zbuf, acc_sh, sem0, sem1, semi_s, semi_d):
        c = lax.axis_index("c")
        s = lax.axis_index("s")

        _zero_fill(zbuf, dh, dt)

        def zero_blk(j, _):
            pltpu.sync_copy(zbuf, acc_sh.at[pl.ds(s * TPN + j * ZR, ZR)])
            return 0

        lax.fori_loop(0, TPN // ZR, zero_blk, 0)
        plsc.subcore_barrier()

        def body(hp_hbm, out_hbm):
            _mp_pipeline(hp_hbm, acc_sh, srcr, dstr, rows0, rows1,
                         sem0, sem1, semi_s, semi_d, src_hbm, dst_hbm,
                         s * MP_NB, MP_NB)
            plsc.subcore_barrier()

            def wout(j, _):
                pltpu.sync_copy(acc_sh.at[pl.ds(s * TPN + j * ZR, ZR)], zbuf)
                pltpu.sync_copy(zbuf, out_hbm.at[pl.ds(s * TPN + j * ZR, ZR)])
                return 0

            lax.fori_loop(0, TPN // ZR, wout, 0)

        @pl.when(c == 0)
        def _():
            body(lo_hbm, olo_hbm)

        @pl.when(c == 1)
        def _():
            body(hi_hbm, ohi_hbm)

    return k(hp_lo, hp_hi, src2d, dst2d)


def _sc_msgpass_edges(hp, src2d, dst2d, dt=jnp.float32):
    row_shape = tuple(hp.shape[1:])
    wnb = MP_NB // NC

    @functools.partial(
        pl.kernel,
        out_type=jax.ShapeDtypeStruct((NC * N_PAD,) + row_shape, dt),
        mesh=_mesh,
        compiler_params=_NOTILE,
        scratch_types=[
            pltpu.VMEM((2 * ICH, MP_K), jnp.int32),
            pltpu.VMEM((2 * ICH, MP_K), jnp.int32),
            pltpu.VMEM((MP_K,) + row_shape, dt),
            pltpu.VMEM((MP_K,) + row_shape, dt),
            pltpu.VMEM((ZR,) + row_shape, dt),
            pltpu.VMEM_SHARED((N_PAD,) + row_shape, dt),
            pltpu.SemaphoreType.DMA,
            pltpu.SemaphoreType.DMA,
            pltpu.SemaphoreType.DMA,
            pltpu.SemaphoreType.DMA,
        ],
    )
    def k(hp_hbm, src_hbm, dst_hbm, out_hbm,
          srcr, dstr, rows0, rows1, zbuf, acc_sh,
          sem0, sem1, semi_s, semi_d):
        c = lax.axis_index("c")
        s = lax.axis_index("s")
        w = c * NS + s

        if dt == jnp.bfloat16:
            _zero_fill3(zbuf)
        else:
            _zero_fill(zbuf, row_shape[0], dt)

        def zero_blk(j, _):
            pltpu.sync_copy(zbuf, acc_sh.at[pl.ds(s * TPN + j * ZR, ZR)])
            return 0

        lax.fori_loop(0, TPN // ZR, zero_blk, 0)
        plsc.subcore_barrier()

        _mp_pipeline(hp_hbm, acc_sh, srcr, dstr, rows0, rows1,
                     sem0, sem1, semi_s, semi_d, src_hbm, dst_hbm,
                     w * wnb, wnb)
        plsc.subcore_barrier()

        def wout(j, _):
            pltpu.sync_copy(acc_sh.at[pl.ds(s * TPN + j * ZR, ZR)], zbuf)
            base = pl.multiple_of(c * N_PAD + s * TPN + j * ZR, ZR)
            pltpu.sync_copy(zbuf, out_hbm.at[pl.ds(base, ZR)])
            return 0

        lax.fori_loop(0, TPN // ZR, wout, 0)

    return k(hp, src2d, dst2d)


_R = 1024
_G = N_PAD // _R


def _dinv_block(dega_ref, degb_ref):
    deg = dega_ref[:, 0:1] + degb_ref[:, 0:1] + 1.0
    return lax.rsqrt(deg)


def _tc0_body(x_ref, w_ref, h_ref):
    h_ref[...] = jnp.dot(x_ref[...], w_ref[...],
                         preferred_element_type=jnp.float32)


def _tc1_body(h_ref, dega_ref, degb_ref, lo_ref, hi_ref):
    dinv = _dinv_block(dega_ref, degb_ref)
    hp = h_ref[...] * dinv
    lo_ref[...] = hp[:, : D_HID // 2]
    hi_ref[...] = hp[:, D_HID // 2:]


def _tc2_body(alo_ref, ahi_ref, plo_ref, phi_ref, dega_ref, degb_ref,
              b1_ref, w2_ref, o_ref):
    dinv = _dinv_block(dega_ref, degb_ref)
    zlo = jnp.maximum(dinv * (alo_ref[...] + plo_ref[...])
                      + b1_ref[0:1, : D_HID // 2], 0.0)
    zhi = jnp.maximum(dinv * (ahi_ref[...] + phi_ref[...])
                      + b1_ref[0:1, D_HID // 2:], 0.0)
    z = jnp.concatenate([zlo, zhi], axis=1)
    h2 = jnp.dot(z, w2_ref[...], preferred_element_type=jnp.float32)
    o_ref[...] = h2 * dinv


def _tc3_body(a0_ref, a1_ref, p_ref, dega_ref, degb_ref, b2_ref, out_ref):
    dinv = _dinv_block(dega_ref, degb_ref)
    acc = (a0_ref[:, :D_OUT].astype(jnp.float32)
           + a1_ref[:, :D_OUT].astype(jnp.float32)
           + p_ref[:, :D_OUT].astype(jnp.float32))
    out_ref[...] = dinv * acc + b2_ref[0:1, :]


def _rows_spec(cols):
    return pl.BlockSpec((_R, cols), lambda i: (i, 0))


_DEG_SPECS = [pl.BlockSpec((_R, DEG_W), lambda i: (i, 0)),
              pl.BlockSpec((_R, DEG_W), lambda i: (i + _G, 0))]


def _full_spec(r, c):
    return pl.BlockSpec((r, c), lambda i: (0, 0))


def kernel(x, edge_index, W1, b1, W2, b2):
    src = edge_index[0].astype(jnp.int32)
    dst = edge_index[1].astype(jnp.int32)
    src_mp = src.reshape(E // MP_K, MP_K)
    dst_mp = dst.reshape(E // MP_K, MP_K)
    xp = jnp.pad(x, ((0, N_PAD - N), (0, 0)))
    b1r = b1.reshape(1, D_HID)
    b2r = b2.reshape(1, D_OUT)

    deg_hist = _sc_degree(dst_mp)

    h1 = pl.pallas_call(
        _tc0_body,
        grid=(_G,),
        in_specs=[_rows_spec(D_IN), _full_spec(D_IN, D_HID)],
        out_specs=_rows_spec(D_HID),
        out_shape=jax.ShapeDtypeStruct((N_PAD, D_HID), jnp.float32),
    )(xp, W1)

    h1p_lo, h1p_hi = pl.pallas_call(
        _tc1_body,
        grid=(_G,),
        in_specs=[_rows_spec(D_HID)] + _DEG_SPECS,
        out_specs=[_rows_spec(D_HID // 2)] * 2,
        out_shape=[jax.ShapeDtypeStruct((N_PAD, D_HID // 2), jnp.float32)] * 2,
    )(h1, deg_hist, deg_hist)

    acc1_lo, acc1_hi = _sc_msgpass(h1p_lo, h1p_hi, src_mp, dst_mp, D_HID // 2)

    h2p = pl.pallas_call(
        _tc2_body,
        grid=(_G,),
        in_specs=[
            _rows_spec(D_HID // 2), _rows_spec(D_HID // 2),
            _rows_spec(D_HID // 2), _rows_spec(D_HID // 2),
        ] + _DEG_SPECS + [
            _full_spec(1, D_HID),
            _full_spec(D_HID, D_OUT),
        ],
        out_specs=_rows_spec(D_OUT),
        out_shape=jax.ShapeDtypeStruct((N_PAD, D_OUT), jnp.float32),
    )(acc1_lo, acc1_hi, h1p_lo, h1p_hi, deg_hist, deg_hist, b1r, W2)

    acc2 = _sc_msgpass_edges(h2p, src_mp, dst_mp)

    out = pl.pallas_call(
        _tc3_body,
        grid=(_G,),
        in_specs=[
            pl.BlockSpec((_R, D_OUT), lambda i: (i, 0)),
            pl.BlockSpec((_R, D_OUT), lambda i: (i + _G, 0)),
            _rows_spec(D_OUT),
        ] + _DEG_SPECS + [
            _full_spec(1, D_OUT),
        ],
        out_specs=_rows_spec(D_OUT),
        out_shape=jax.ShapeDtypeStruct((N_PAD, D_OUT), jnp.float32),
    )(acc2, acc2, h2p, deg_hist, deg_hist, b2r)

    return out[:N]


# speedup vs baseline: 21.3326x; 1.0330x over previous
import functools

import jax
import jax.numpy as jnp
from jax import lax
from jax.experimental import pallas as pl
from jax.experimental.pallas import tpu as pltpu
from jax.experimental.pallas import tpu_sc as plsc

N = 10000
N_PAD = 10240
E = 160000
D_IN = 256
D_HID = 256
D_OUT = 64

NC = 2
NS = 16
TPN = N_PAD // NS
ZR = 64

MP_K = 125
MP_NB = (E // NS) // MP_K

DEG_W = 8

_mesh = plsc.VectorSubcoreMesh(core_axis_name="c", subcore_axis_name="s")
_NOTILE = pltpu.CompilerParams(use_tc_tiling_on_sc=False)


def _sc_degree(dst2d):
    wnb = MP_NB // NC

    @functools.partial(
        pl.kernel,
        out_type=jax.ShapeDtypeStruct((NC * N_PAD, DEG_W), jnp.float32),
        mesh=_mesh,
        compiler_params=_NOTILE,
        scratch_types=[
            pltpu.VMEM((wnb, MP_K), jnp.int32),
            pltpu.VMEM((128, DEG_W), jnp.float32),
            pltpu.VMEM((ZR, DEG_W), jnp.float32),
            pltpu.VMEM_SHARED((N_PAD, DEG_W), jnp.float32),
        ],
    )
    def k(dst_hbm, out_hbm, dstv, ones, zbuf, hist_sh):
        c = lax.axis_index("c")
        s = lax.axis_index("s")
        w = c * NS + s

        def fill_ones(r, _):
            ones[pl.ds(r * 16, 16), pl.ds(0, DEG_W)] = (
                jnp.ones((16, DEG_W), jnp.float32))
            return 0

        lax.fori_loop(0, 128 // 16, fill_ones, 0)

        def fill_zero(r, _):
            zbuf[pl.ds(r * 16, 16), pl.ds(0, DEG_W)] = (
                jnp.zeros((16, DEG_W), jnp.float32))
            return 0

        lax.fori_loop(0, ZR // 16, fill_zero, 0)

        def zero_blk(j, _):
            pltpu.sync_copy(zbuf, hist_sh.at[pl.ds(s * TPN + j * ZR, ZR)])
            return 0

        lax.fori_loop(0, TPN // ZR, zero_blk, 0)
        pltpu.sync_copy(dst_hbm.at[pl.ds(w * wnb, wnb)], dstv)
        plsc.subcore_barrier()

        def scat(b, _):
            pltpu.sync_copy(ones.at[pl.ds(0, MP_K)],
                            hist_sh.at[dstv.at[b]], add=True)
            return 0

        lax.fori_loop(0, wnb, scat, 0)
        plsc.subcore_barrier()

        def wout(j, _):
            pltpu.sync_copy(hist_sh.at[pl.ds(s * TPN + j * ZR, ZR)], zbuf)
            base = pl.multiple_of(c * N_PAD + s * TPN + j * ZR, ZR)
            pltpu.sync_copy(zbuf, out_hbm.at[pl.ds(base, ZR)])
            return 0

        lax.fori_loop(0, TPN // ZR, wout, 0)

    return k(dst2d)


ICH = 8


def _ring_row(b):
    return ((b // ICH) % 2) * ICH + (b % ICH)


def _mp_pipeline(hp_hbm, acc_sh, srcr, dstr, rows0, rows1,
                 sem0, sem1, semi_s, semi_d, src_hbm, dst_hbm,
                 base_row, wrows):
    pltpu.sync_copy(src_hbm.at[pl.ds(base_row, ICH)], srcr.at[pl.ds(0, ICH)])
    pltpu.sync_copy(dst_hbm.at[pl.ds(base_row, ICH)], dstr.at[pl.ds(0, ICH)])
    pltpu.async_copy(hp_hbm.at[srcr.at[0]], rows0, sem0)

    def pair(p, _):
        b0 = 2 * p
        b1 = b0 + 1

        @pl.when(jnp.logical_and(b0 % ICH == 0, b0 + ICH < wrows))
        def _():
            cn = b0 // ICH + 1
            off = pl.multiple_of((cn % 2) * ICH, ICH)
            pltpu.async_copy(src_hbm.at[pl.ds(base_row + cn * ICH, ICH)],
                             srcr.at[pl.ds(off, ICH)], semi_s)
            pltpu.async_copy(dst_hbm.at[pl.ds(base_row + cn * ICH, ICH)],
                             dstr.at[pl.ds(off, ICH)], semi_d)

        rr0 = _ring_row(b0)
        rr1 = _ring_row(b1)
        pltpu.async_copy(hp_hbm.at[srcr.at[rr1]], rows1, sem1)
        pltpu.make_async_copy(hp_hbm.at[srcr.at[rr0]], rows0, sem0).wait()
        pltpu.sync_copy(rows0, acc_sh.at[dstr.at[rr0]], add=True)

        @pl.when(b0 + 2 < wrows)
        def _():
            @pl.when((b0 + 2) % ICH == 0)
            def _():
                pltpu.make_async_copy(
                    src_hbm.at[pl.ds(base_row, ICH)],
                    srcr.at[pl.ds(0, ICH)], semi_s).wait()
                pltpu.make_async_copy(
                    dst_hbm.at[pl.ds(base_row, ICH)],
                    dstr.at[pl.ds(0, ICH)], semi_d).wait()

            pltpu.async_copy(hp_hbm.at[srcr.at[_ring_row(b0 + 2)]],
                             rows0, sem0)

        pltpu.make_async_copy(hp_hbm.at[srcr.at[rr1]], rows1, sem1).wait()
        pltpu.sync_copy(rows1, acc_sh.at[dstr.at[rr1]], add=True)
        return 0

    lax.fori_loop(0, wrows // 2, pair, 0)


def _zero_fill(zbuf, dh, dt=jnp.float32):
    if dt == jnp.bfloat16:
        def fill_zero(r, _):
            def col(j, _):
                zbuf[pl.ds(2 * r, 2), pl.ds(j * 16, 16)] = (
                    jnp.zeros((2, 16), dt))
                return 0
            return lax.fori_loop(0, dh // 16, col, 0)

        lax.fori_loop(0, ZR // 2, fill_zero, 0)
    else:
        def fill_zero(r, _):
            def col(j, _):
                zbuf[r, pl.ds(j * 16, 16)] = jnp.zeros((16,), dt)
                return 0
            return lax.fori_loop(0, dh // 16, col, 0)

        lax.fori_loop(0, ZR, fill_zero, 0)


def _zero_fill3(zbuf):
    def fill_zero(r, _):
        def col(j, _):
            zbuf[r, pl.ds(0, 2), pl.ds(j * 16, 16)] = (
                jnp.zeros((2, 16), jnp.bfloat16))
            return 0
        return lax.fori_loop(0, 8, col, 0)

    lax.fori_loop(0, ZR, fill_zero, 0)


def _sc_msgpass(hp_lo, hp_hi, src2d, dst2d, dh, dt=jnp.float32):
    @functools.partial(
        pl.kernel,
        out_type=[jax.ShapeDtypeStruct((N_PAD, dh), dt)] * 2,
        mesh=_mesh,
        scratch_types=[
            pltpu.VMEM((2 * ICH, MP_K), jnp.int32),
            pltpu.VMEM((2 * ICH, MP_K), jnp.int32),
            pltpu.VMEM((MP_K, dh), dt),
            pltpu.VMEM((MP_K, dh), dt),
            pltpu.VMEM((ZR, dh), dt),
            pltpu.VMEM_SHARED((N_PAD, dh), dt),
            pltpu.SemaphoreType.DMA,
            pltpu.SemaphoreType.DMA,
            pltpu.SemaphoreType.DMA,
            pltpu.SemaphoreType.DMA,
        ],
    )
    def k(lo_hbm, hi_hbm, src_hbm, dst_hbm, olo_hbm, ohi_hbm,
          srcr, dstr, rows0, rows1, zbuf, acc_sh, sem0, sem1, semi_s, semi_d):
        c = lax.axis_index("c")
        s = lax.axis_index("s")

        _zero_fill(zbuf, dh, dt)

        def zero_blk(j, _):
            pltpu.sync_copy(zbuf, acc_sh.at[pl.ds(s * TPN + j * ZR, ZR)])
            return 0

        lax.fori_loop(0, TPN // ZR, zero_blk, 0)
        plsc.subcore_barrier()

        def body(hp_hbm, out_hbm):
            _mp_pipeline(hp_hbm, acc_sh, srcr, dstr, rows0, rows1,
                         sem0, sem1, semi_s, semi_d, src_hbm, dst_hbm,
                         s * MP_NB, MP_NB)
            plsc.subcore_barrier()

            def wout(j, _):
                pltpu.sync_copy(acc_sh.at[pl.ds(s * TPN + j * ZR, ZR)], zbuf)
                pltpu.sync_copy(zbuf, out_hbm.at[pl.ds(s * TPN + j * ZR, ZR)])
                return 0

            lax.fori_loop(0, TPN // ZR, wout, 0)

        @pl.when(c == 0)
        def _():
            body(lo_hbm, olo_hbm)

        @pl.when(c == 1)
        def _():
            body(hi_hbm, ohi_hbm)

    return k(hp_lo, hp_hi, src2d, dst2d)


def _sc_msgpass_edges(hp, src2d, dst2d, dt=jnp.float32):
    row_shape = tuple(hp.shape[1:])
    wnb = MP_NB // NC

    @functools.partial(
        pl.kernel,
        out_type=jax.ShapeDtypeStruct((NC * N_PAD,) + row_shape, dt),
        mesh=_mesh,
        compiler_params=_NOTILE,
        scratch_types=[
            pltpu.VMEM((2 * ICH, MP_K), jnp.int32),
            pltpu.VMEM((2 * ICH, MP_K), jnp.int32),
            pltpu.VMEM((MP_K,) + row_shape, dt),
            pltpu.VMEM((MP_K,) + row_shape, dt),
            pltpu.VMEM((ZR,) + row_shape, dt),
            pltpu.VMEM_SHARED((N_PAD,) + row_shape, dt),
            pltpu.SemaphoreType.DMA,
            pltpu.SemaphoreType.DMA,
            pltpu.SemaphoreType.DMA,
            pltpu.SemaphoreType.DMA,
        ],
    )
    def k(hp_hbm, src_hbm, dst_hbm, out_hbm,
          srcr, dstr, rows0, rows1, zbuf, acc_sh,
          sem0, sem1, semi_s, semi_d):
        c = lax.axis_index("c")
        s = lax.axis_index("s")
        w = c * NS + s

        if dt == jnp.bfloat16:
            _zero_fill3(zbuf)
        else:
            _zero_fill(zbuf, row_shape[0], dt)

        def zero_blk(j, _):
            pltpu.sync_copy(zbuf, acc_sh.at[pl.ds(s * TPN + j * ZR, ZR)])
            return 0

        lax.fori_loop(0, TPN // ZR, zero_blk, 0)
        plsc.subcore_barrier()

        _mp_pipeline(hp_hbm, acc_sh, srcr, dstr, rows0, rows1,
                     sem0, sem1, semi_s, semi_d, src_hbm, dst_hbm,
                     w * wnb, wnb)
        plsc.subcore_barrier()

        def wout(j, _):
            pltpu.sync_copy(acc_sh.at[pl.ds(s * TPN + j * ZR, ZR)], zbuf)
            base = pl.multiple_of(c * N_PAD + s * TPN + j * ZR, ZR)
            pltpu.sync_copy(zbuf, out_hbm.at[pl.ds(base, ZR)])
            return 0

        lax.fori_loop(0, TPN // ZR, wout, 0)

    return k(hp, src2d, dst2d)


_R = 1000
_G = N // _R


def _dinv_block(dega_ref, degb_ref):
    deg = dega_ref[0, :, 0:1] + degb_ref[0, :, 0:1] + 1.0
    return lax.rsqrt(deg)


def _tc0_body(x_ref, w_ref, h_ref):
    h_ref[...] = jnp.dot(x_ref[...], w_ref[...],
                         preferred_element_type=jnp.float32)


def _tc1_body(h_ref, dega_ref, degb_ref, lo_ref, hi_ref):
    dinv = _dinv_block(dega_ref, degb_ref)
    hp = h_ref[...] * dinv
    lo_ref[...] = hp[:, : D_HID // 2]
    hi_ref[...] = hp[:, D_HID // 2:]


def _tc2_body(alo_ref, ahi_ref, plo_ref, phi_ref, dega_ref, degb_ref,
              b1_ref, w2_ref, o_ref):
    dinv = _dinv_block(dega_ref, degb_ref)
    zlo = jnp.maximum(dinv * (alo_ref[...] + plo_ref[...])
                      + b1_ref[0:1, : D_HID // 2], 0.0)
    zhi = jnp.maximum(dinv * (ahi_ref[...] + phi_ref[...])
                      + b1_ref[0:1, D_HID // 2:], 0.0)
    z = jnp.concatenate([zlo, zhi], axis=1)
    h2 = jnp.dot(z, w2_ref[...], preferred_element_type=jnp.float32)
    o_ref[...] = h2 * dinv


def _tc3_body(a0_ref, a1_ref, p_ref, dega_ref, degb_ref, b2_ref, out_ref):
    dinv = _dinv_block(dega_ref, degb_ref)
    acc = a0_ref[0] + a1_ref[0] + p_ref[...]
    out_ref[...] = dinv * acc + b2_ref[0:1, :]


def _rows_spec(cols):
    return pl.BlockSpec((_R, cols), lambda i: (i, 0))


_DEG_SPECS = [pl.BlockSpec((1, _R, DEG_W), lambda i: (0, i, 0)),
              pl.BlockSpec((1, _R, DEG_W), lambda i: (1, i, 0))]


def _full_spec(r, c):
    return pl.BlockSpec((r, c), lambda i: (0, 0))


def kernel(x, edge_index, W1, b1, W2, b2):
    src = edge_index[0].astype(jnp.int32)
    dst = edge_index[1].astype(jnp.int32)
    src_mp = src.reshape(E // MP_K, MP_K)
    dst_mp = dst.reshape(E // MP_K, MP_K)
    b1r = b1.reshape(1, D_HID)
    b2r = b2.reshape(1, D_OUT)

    deg3 = _sc_degree(dst_mp).reshape(NC, N_PAD, DEG_W)

    h1 = pl.pallas_call(
        _tc0_body,
        grid=(_G,),
        in_specs=[_rows_spec(D_IN), _full_spec(D_IN, D_HID)],
        out_specs=_rows_spec(D_HID),
        out_shape=jax.ShapeDtypeStruct((N, D_HID), jnp.float32),
    )(x, W1)

    h1p_lo, h1p_hi = pl.pallas_call(
        _tc1_body,
        grid=(_G,),
        in_specs=[_rows_spec(D_HID)] + _DEG_SPECS,
        out_specs=[_rows_spec(D_HID // 2)] * 2,
        out_shape=[jax.ShapeDtypeStruct((N, D_HID // 2), jnp.float32)] * 2,
    )(h1, deg3, deg3)

    acc1_lo, acc1_hi = _sc_msgpass(h1p_lo, h1p_hi, src_mp, dst_mp, D_HID // 2)

    h2p = pl.pallas_call(
        _tc2_body,
        grid=(_G,),
        in_specs=[
            _rows_spec(D_HID // 2), _rows_spec(D_HID // 2),
            _rows_spec(D_HID // 2), _rows_spec(D_HID // 2),
        ] + _DEG_SPECS + [
            _full_spec(1, D_HID),
            _full_spec(D_HID, D_OUT),
        ],
        out_specs=_rows_spec(D_OUT),
        out_shape=jax.ShapeDtypeStruct((N, D_OUT), jnp.float32),
    )(acc1_lo, acc1_hi, h1p_lo, h1p_hi, deg3, deg3, b1r, W2)

    acc2 = _sc_msgpass_edges(h2p, src_mp, dst_mp).reshape(NC, N_PAD, D_OUT)

    out = pl.pallas_call(
        _tc3_body,
        grid=(_G,),
        in_specs=[
            pl.BlockSpec((1, _R, D_OUT), lambda i: (0, i, 0)),
            pl.BlockSpec((1, _R, D_OUT), lambda i: (1, i, 0)),
            _rows_spec(D_OUT),
        ] + _DEG_SPECS + [
            _full_spec(1, D_OUT),
        ],
        out_specs=_rows_spec(D_OUT),
        out_shape=jax.ShapeDtypeStruct((N, D_OUT), jnp.float32),
    )(acc2, acc2, h2p, deg3, deg3, b2r)

    return out


# speedup vs baseline: 21.7548x; 1.0198x over previous
import functools

import jax
import jax.numpy as jnp
from jax import lax
from jax.experimental import pallas as pl
from jax.experimental.pallas import tpu as pltpu
from jax.experimental.pallas import tpu_sc as plsc

N = 10000
N_PAD = 10240
E = 160000
D_IN = 256
D_HID = 256
D_OUT = 64

NC = 2
NS = 16
TPN = N_PAD // NS
ZR = 64

MP_K = 125
MP_NB = (E // NS) // MP_K

DEG_W = 8

_mesh = plsc.VectorSubcoreMesh(core_axis_name="c", subcore_axis_name="s")
_NOTILE = pltpu.CompilerParams(use_tc_tiling_on_sc=False)


def _sc_degree(dst2d):
    wnb = MP_NB // NC

    @functools.partial(
        pl.kernel,
        out_type=jax.ShapeDtypeStruct((NC * N_PAD, DEG_W), jnp.float32),
        mesh=_mesh,
        compiler_params=_NOTILE,
        scratch_types=[
            pltpu.VMEM((wnb, MP_K), jnp.int32),
            pltpu.VMEM((128, DEG_W), jnp.float32),
            pltpu.VMEM((ZR, DEG_W), jnp.float32),
            pltpu.VMEM_SHARED((N_PAD, DEG_W), jnp.float32),
        ],
    )
    def k(dst_hbm, out_hbm, dstv, ones, zbuf, hist_sh):
        c = lax.axis_index("c")
        s = lax.axis_index("s")
        w = c * NS + s

        def fill_ones(r, _):
            ones[pl.ds(r * 16, 16), pl.ds(0, DEG_W)] = (
                jnp.ones((16, DEG_W), jnp.float32))
            return 0

        lax.fori_loop(0, 128 // 16, fill_ones, 0)

        def fill_zero(r, _):
            zbuf[pl.ds(r * 16, 16), pl.ds(0, DEG_W)] = (
                jnp.zeros((16, DEG_W), jnp.float32))
            return 0

        lax.fori_loop(0, ZR // 16, fill_zero, 0)

        def zero_blk(j, _):
            pltpu.sync_copy(zbuf, hist_sh.at[pl.ds(s * TPN + j * ZR, ZR)])
            return 0

        lax.fori_loop(0, TPN // ZR, zero_blk, 0)
        pltpu.sync_copy(dst_hbm.at[pl.ds(w * wnb, wnb)], dstv)
        plsc.subcore_barrier()

        def scat(b, _):
            pltpu.sync_copy(ones.at[pl.ds(0, MP_K)],
                            hist_sh.at[dstv.at[b]], add=True)
            return 0

        lax.fori_loop(0, wnb, scat, 0)
        plsc.subcore_barrier()

        def wout(j, _):
            pltpu.sync_copy(hist_sh.at[pl.ds(s * TPN + j * ZR, ZR)], zbuf)
            base = pl.multiple_of(c * N_PAD + s * TPN + j * ZR, ZR)
            pltpu.sync_copy(zbuf, out_hbm.at[pl.ds(base, ZR)])
            return 0

        lax.fori_loop(0, TPN // ZR, wout, 0)

    return k(dst2d)


ICH = 8


def _ring_row(b):
    return ((b // ICH) % 2) * ICH + (b % ICH)


def _mp_pipeline(hp_hbm, acc_sh, srcr, dstr, rows0, rows1,
                 sem0, sem1, semi_s, semi_d, src_hbm, dst_hbm,
                 base_row, wrows):
    pltpu.sync_copy(src_hbm.at[pl.ds(base_row, ICH)], srcr.at[pl.ds(0, ICH)])
    pltpu.sync_copy(dst_hbm.at[pl.ds(base_row, ICH)], dstr.at[pl.ds(0, ICH)])
    pltpu.async_copy(hp_hbm.at[srcr.at[0]], rows0, sem0)

    def pair(p, _):
        b0 = 2 * p
        b1 = b0 + 1

        @pl.when(jnp.logical_and(b0 % ICH == 0, b0 + ICH < wrows))
        def _():
            cn = b0 // ICH + 1
            off = pl.multiple_of((cn % 2) * ICH, ICH)
            pltpu.async_copy(src_hbm.at[pl.ds(base_row + cn * ICH, ICH)],
                             srcr.at[pl.ds(off, ICH)], semi_s)
            pltpu.async_copy(dst_hbm.at[pl.ds(base_row + cn * ICH, ICH)],
                             dstr.at[pl.ds(off, ICH)], semi_d)

        rr0 = _ring_row(b0)
        rr1 = _ring_row(b1)
        pltpu.async_copy(hp_hbm.at[srcr.at[rr1]], rows1, sem1)
        pltpu.make_async_copy(hp_hbm.at[srcr.at[rr0]], rows0, sem0).wait()
        pltpu.sync_copy(rows0, acc_sh.at[dstr.at[rr0]], add=True)

        @pl.when(b0 + 2 < wrows)
        def _():
            @pl.when((b0 + 2) % ICH == 0)
            def _():
                pltpu.make_async_copy(
                    src_hbm.at[pl.ds(base_row, ICH)],
                    srcr.at[pl.ds(0, ICH)], semi_s).wait()
                pltpu.make_async_copy(
                    dst_hbm.at[pl.ds(base_row, ICH)],
                    dstr.at[pl.ds(0, ICH)], semi_d).wait()

            pltpu.async_copy(hp_hbm.at[srcr.at[_ring_row(b0 + 2)]],
                             rows0, sem0)

        pltpu.make_async_copy(hp_hbm.at[srcr.at[rr1]], rows1, sem1).wait()
        pltpu.sync_copy(rows1, acc_sh.at[dstr.at[rr1]], add=True)
        return 0

    lax.fori_loop(0, wrows // 2, pair, 0)


def _zero_fill(zbuf, dh, dt=jnp.float32):
    if dt == jnp.bfloat16:
        def fill_zero(r, _):
            def col(j, _):
                zbuf[pl.ds(2 * r, 2), pl.ds(j * 16, 16)] = (
                    jnp.zeros((2, 16), dt))
                return 0
            return lax.fori_loop(0, dh // 16, col, 0)

        lax.fori_loop(0, ZR // 2, fill_zero, 0)
    else:
        def fill_zero(r, _):
            def col(j, _):
                zbuf[r, pl.ds(j * 16, 16)] = jnp.zeros((16,), dt)
                return 0
            return lax.fori_loop(0, dh // 16, col, 0)

        lax.fori_loop(0, ZR, fill_zero, 0)


def _zero_fill3(zbuf):
    def fill_zero(r, _):
        def col(j, _):
            zbuf[r, pl.ds(0, 2), pl.ds(j * 16, 16)] = (
                jnp.zeros((2, 16), jnp.bfloat16))
            return 0
        return lax.fori_loop(0, 8, col, 0)

    lax.fori_loop(0, ZR, fill_zero, 0)


def _sc_msgpass(hp_lo, hp_hi, src2d, dst2d, dh, dt=jnp.float32):
    @functools.partial(
        pl.kernel,
        out_type=[jax.ShapeDtypeStruct((N_PAD, dh), dt)] * 2,
        mesh=_mesh,
        scratch_types=[
            pltpu.VMEM((2 * ICH, MP_K), jnp.int32),
            pltpu.VMEM((2 * ICH, MP_K), jnp.int32),
            pltpu.VMEM((MP_K, dh), dt),
            pltpu.VMEM((MP_K, dh), dt),
            pltpu.VMEM((ZR, dh), dt),
            pltpu.VMEM_SHARED((N_PAD, dh), dt),
            pltpu.SemaphoreType.DMA,
            pltpu.SemaphoreType.DMA,
            pltpu.SemaphoreType.DMA,
            pltpu.SemaphoreType.DMA,
        ],
    )
    def k(lo_hbm, hi_hbm, src_hbm, dst_hbm, olo_hbm, ohi_hbm,
          srcr, dstr, rows0, rows1, zbuf, acc_sh, sem0, sem1, semi_s, semi_d):
        c = lax.axis_index("c")
        s = lax.axis_index("s")

        _zero_fill(zbuf, dh, dt)

        def zero_blk(j, _):
            pltpu.sync_copy(zbuf, acc_sh.at[pl.ds(s * TPN + j * ZR, ZR)])
            return 0

        lax.fori_loop(0, TPN // ZR, zero_blk, 0)
        plsc.subcore_barrier()

        def body(hp_hbm, out_hbm):
            _mp_pipeline(hp_hbm, acc_sh, srcr, dstr, rows0, rows1,
                         sem0, sem1, semi_s, semi_d, src_hbm, dst_hbm,
                         s * MP_NB, MP_NB)
            plsc.subcore_barrier()

            def wout(j, _):
                pltpu.sync_copy(acc_sh.at[pl.ds(s * TPN + j * ZR, ZR)], zbuf)
                pltpu.sync_copy(zbuf, out_hbm.at[pl.ds(s * TPN + j * ZR, ZR)])
                return 0

            lax.fori_loop(0, TPN // ZR, wout, 0)

        @pl.when(c == 0)
        def _():
            body(lo_hbm, olo_hbm)

        @pl.when(c == 1)
        def _():
            body(hi_hbm, ohi_hbm)

    return k(hp_lo, hp_hi, src2d, dst2d)


def _mp_pipeline3(hp_hbm, acc_sh, srcv, dstv, rows, gsems, tsems,
                  base_row, wrows):
    def g_issue(b, j):
        pltpu.async_copy(hp_hbm.at[srcv.at[base_row + b]], rows[j], gsems[j])

    def g_wait(b, j):
        pltpu.make_async_copy(hp_hbm.at[srcv.at[base_row + b]], rows[j],
                              gsems[j]).wait()

    def s_issue(b, j):
        pltpu.async_copy(rows[j], acc_sh.at[dstv.at[base_row + b]], tsems[j],
                         add=True)

    def s_wait(b, j):
        pltpu.make_async_copy(rows[j], acc_sh.at[dstv.at[base_row + b]],
                              tsems[j]).wait()

    g_issue(0, 0)
    g_issue(1, 1)

    def tri(g, _):
        for j in range(3):
            b = 3 * g + j

            @pl.when(b < wrows)
            def _():
                g_wait(b, j)
                s_issue(b, j)
                nb = b + 2
                j2 = (j + 2) % 3

                @pl.when(nb < wrows)
                def _():
                    @pl.when(nb >= 3)
                    def _():
                        s_wait(nb - 3, j2)

                    g_issue(nb, j2)
        return 0

    lax.fori_loop(0, (wrows + 2) // 3, tri, 0)
    for d in range(1, 4):
        b = wrows - d
        if b >= 0:
            s_wait(b, b % 3)


def _sc_msgpass_edges(hp, src2d, dst2d, dt=jnp.float32):
    row_shape = tuple(hp.shape[1:])
    wnb = MP_NB // NC

    @functools.partial(
        pl.kernel,
        out_type=jax.ShapeDtypeStruct((NC * N_PAD,) + row_shape, dt),
        mesh=_mesh,
        compiler_params=_NOTILE,
        scratch_types=[
            pltpu.VMEM((MP_NB // NC, MP_K), jnp.int32),
            pltpu.VMEM((MP_NB // NC, MP_K), jnp.int32),
            pltpu.VMEM((MP_K,) + row_shape, dt),
            pltpu.VMEM((MP_K,) + row_shape, dt),
            pltpu.VMEM((MP_K,) + row_shape, dt),
            pltpu.VMEM((ZR,) + row_shape, dt),
            pltpu.VMEM_SHARED((N_PAD,) + row_shape, dt),
            pltpu.SemaphoreType.DMA,
            pltpu.SemaphoreType.DMA,
            pltpu.SemaphoreType.DMA,
            pltpu.SemaphoreType.DMA,
            pltpu.SemaphoreType.DMA,
            pltpu.SemaphoreType.DMA,
        ],
    )
    def k(hp_hbm, src_hbm, dst_hbm, out_hbm,
          srcv, dstv, rows0, rows1, rows2, zbuf, acc_sh,
          g0, g1, g2, t0, t1, t2):
        c = lax.axis_index("c")
        s = lax.axis_index("s")
        w = c * NS + s

        if dt == jnp.bfloat16:
            _zero_fill3(zbuf)
        else:
            _zero_fill(zbuf, row_shape[0], dt)

        def zero_blk(j, _):
            pltpu.sync_copy(zbuf, acc_sh.at[pl.ds(s * TPN + j * ZR, ZR)])
            return 0

        lax.fori_loop(0, TPN // ZR, zero_blk, 0)
        pltpu.sync_copy(src_hbm.at[pl.ds(w * wnb, wnb)], srcv)
        pltpu.sync_copy(dst_hbm.at[pl.ds(w * wnb, wnb)], dstv)
        plsc.subcore_barrier()

        _mp_pipeline3(hp_hbm, acc_sh, srcv, dstv, [rows0, rows1, rows2],
                      [g0, g1, g2], [t0, t1, t2], 0, wnb)
        plsc.subcore_barrier()

        def wout(j, _):
            pltpu.sync_copy(acc_sh.at[pl.ds(s * TPN + j * ZR, ZR)], zbuf)
            base = pl.multiple_of(c * N_PAD + s * TPN + j * ZR, ZR)
            pltpu.sync_copy(zbuf, out_hbm.at[pl.ds(base, ZR)])
            return 0

        lax.fori_loop(0, TPN // ZR, wout, 0)

    return k(hp, src2d, dst2d)


_R = 1000
_G = N // _R


def _dinv_block(dega_ref, degb_ref):
    deg = dega_ref[0, :, 0:1] + degb_ref[0, :, 0:1] + 1.0
    return lax.rsqrt(deg)


def _tc0_body(x_ref, w_ref, h_ref):
    h_ref[...] = jnp.dot(x_ref[...], w_ref[...],
                         preferred_element_type=jnp.float32)


def _tc1_body(h_ref, dega_ref, degb_ref, lo_ref, hi_ref):
    dinv = _dinv_block(dega_ref, degb_ref)
    hp = h_ref[...] * dinv
    lo_ref[...] = hp[:, : D_HID // 2]
    hi_ref[...] = hp[:, D_HID // 2:]


def _tc2_body(alo_ref, ahi_ref, plo_ref, phi_ref, dega_ref, degb_ref,
              b1_ref, w2_ref, o_ref):
    dinv = _dinv_block(dega_ref, degb_ref)
    zlo = jnp.maximum(dinv * (alo_ref[...] + plo_ref[...])
                      + b1_ref[0:1, : D_HID // 2], 0.0)
    zhi = jnp.maximum(dinv * (ahi_ref[...] + phi_ref[...])
                      + b1_ref[0:1, D_HID // 2:], 0.0)
    z = jnp.concatenate([zlo, zhi], axis=1)
    h2 = jnp.dot(z, w2_ref[...], preferred_element_type=jnp.float32)
    o_ref[...] = h2 * dinv


def _tc3_body(a0_ref, a1_ref, p_ref, dega_ref, degb_ref, b2_ref, out_ref):
    dinv = _dinv_block(dega_ref, degb_ref)
    acc = a0_ref[0] + a1_ref[0] + p_ref[...]
    out_ref[...] = dinv * acc + b2_ref[0:1, :]


def _rows_spec(cols):
    return pl.BlockSpec((_R, cols), lambda i: (i, 0))


_DEG_SPECS = [pl.BlockSpec((1, _R, DEG_W), lambda i: (0, i, 0)),
              pl.BlockSpec((1, _R, DEG_W), lambda i: (1, i, 0))]


def _full_spec(r, c):
    return pl.BlockSpec((r, c), lambda i: (0, 0))


def kernel(x, edge_index, W1, b1, W2, b2):
    src = edge_index[0].astype(jnp.int32)
    dst = edge_index[1].astype(jnp.int32)
    src_mp = src.reshape(E // MP_K, MP_K)
    dst_mp = dst.reshape(E // MP_K, MP_K)
    b1r = b1.reshape(1, D_HID)
    b2r = b2.reshape(1, D_OUT)

    deg3 = _sc_degree(dst_mp).reshape(NC, N_PAD, DEG_W)

    h1 = pl.pallas_call(
        _tc0_body,
        grid=(_G,),
        in_specs=[_rows_spec(D_IN), _full_spec(D_IN, D_HID)],
        out_specs=_rows_spec(D_HID),
        out_shape=jax.ShapeDtypeStruct((N, D_HID), jnp.float32),
    )(x, W1)

    h1p_lo, h1p_hi = pl.pallas_call(
        _tc1_body,
        grid=(_G,),
        in_specs=[_rows_spec(D_HID)] + _DEG_SPECS,
        out_specs=[_rows_spec(D_HID // 2)] * 2,
        out_shape=[jax.ShapeDtypeStruct((N, D_HID // 2), jnp.float32)] * 2,
    )(h1, deg3, deg3)

    acc1_lo, acc1_hi = _sc_msgpass(h1p_lo, h1p_hi, src_mp, dst_mp, D_HID // 2)

    h2p = pl.pallas_call(
        _tc2_body,
        grid=(_G,),
        in_specs=[
            _rows_spec(D_HID // 2), _rows_spec(D_HID // 2),
            _rows_spec(D_HID // 2), _rows_spec(D_HID // 2),
        ] + _DEG_SPECS + [
            _full_spec(1, D_HID),
            _full_spec(D_HID, D_OUT),
        ],
        out_specs=_rows_spec(D_OUT),
        out_shape=jax.ShapeDtypeStruct((N, D_OUT), jnp.float32),
    )(acc1_lo, acc1_hi, h1p_lo, h1p_hi, deg3, deg3, b1r, W2)

    acc2 = _sc_msgpass_edges(h2p, src_mp, dst_mp).reshape(NC, N_PAD, D_OUT)

    out = pl.pallas_call(
        _tc3_body,
        grid=(_G,),
        in_specs=[
            pl.BlockSpec((1, _R, D_OUT), lambda i: (0, i, 0)),
            pl.BlockSpec((1, _R, D_OUT), lambda i: (1, i, 0)),
            _rows_spec(D_OUT),
        ] + _DEG_SPECS + [
            _full_spec(1, D_OUT),
        ],
        out_specs=_rows_spec(D_OUT),
        out_shape=jax.ShapeDtypeStruct((N, D_OUT), jnp.float32),
    )(acc2, acc2, h2p, deg3, deg3, b2r)

    return out
